# baseline (speedup 1.0000x reference)
import functools

import jax
import jax.numpy as jnp
import numpy as np
from jax import lax
from jax.experimental import pallas as pl
from jax.experimental.pallas import tpu as pltpu
from jax.experimental.pallas import tpu_sc as plsc

N = 10000
NP = 10240
E = 320000
EP = 327680
G = 64
D = 128
NW = 32
NB = 1280
EB = 1024
NFREQ = 10

f32 = jnp.float32
i32 = jnp.int32

def _mesh():
    return plsc.VectorSubcoreMesh(core_axis_name="c", subcore_axis_name="s")


def _sc_gather(table, idx2d, d):
    mrows = idx2d.shape[0]
    m = mrows * 128
    nch = mrows // NW
    nbuf = 4
    nh = nch // nbuf
    per = nch * 128

    @functools.partial(
        pl.kernel,
        out_type=jax.ShapeDtypeStruct((m, d), f32),
        mesh=_mesh(),
        scratch_types=[
            pltpu.VMEM((nch, 128), i32),
        ] + [pltpu.VMEM((128, d), f32)] * nbuf
          + [pltpu.SemaphoreType.DMA] * (2 * nbuf),
        compiler_params=pltpu.CompilerParams(use_tc_tiling_on_sc=(d == D)),
    )
    def gk(table_hbm, idx_hbm, out_hbm, idx_v, *bufsem):
        bufs = bufsem[:nbuf]
        gsems = bufsem[nbuf:2 * nbuf]
        osems = bufsem[2 * nbuf:]
        wid = lax.axis_index("s") * 2 + lax.axis_index("c")
        base = wid * per
        pltpu.sync_copy(idx_hbm.at[pl.ds(wid * nch, nch)], idx_v)

        def start_g(c, k):
            pltpu.async_copy(table_hbm.at[idx_v.at[c]], bufs[k], gsems[k])

        def wait_g(k):
            pltpu.make_async_copy(table_hbm.at[idx_v.at[0]], bufs[k],
                                  gsems[k]).wait()

        def start_o(c, k):
            pltpu.async_copy(bufs[k],
                             out_hbm.at[pl.ds(base + c * 128, 128)], osems[k])

        def wait_o(k):
            pltpu.make_async_copy(bufs[k], out_hbm.at[pl.ds(base, 128)],
                                  osems[k]).wait()

        for k in range(nbuf):
            start_g(k, k)

        @pl.loop(0, nh)
        def _(hh):
            c0 = nbuf * hh
            for k in range(nbuf):
                wait_g(k)
                start_o(c0 + k, k)

            @pl.when(hh < nh - 1)
            def _():
                for k in range(nbuf):
                    wait_o(k)
                    start_g(c0 + nbuf + k, k)

            @pl.when(hh == nh - 1)
            def _():
                for k in range(nbuf):
                    wait_o(k)

    return gk(table, idx2d)


def _sc_scatter_add(ef, srcp, zeros_np):
    stripe = NP // 16
    per = EP // NW

    @functools.partial(
        pl.kernel,
        out_type=jax.ShapeDtypeStruct((2, NP, D), f32),
        mesh=_mesh(),
        scratch_types=[
            pltpu.VMEM((EP // NW // 128, 128), i32),
            pltpu.VMEM((128, D), f32),
            pltpu.VMEM((128, D), f32),
            pltpu.SemaphoreType.DMA,
            pltpu.SemaphoreType.DMA,
            pltpu.VMEM_SHARED((NP, D), f32),
        ],
    )
    def sk(ef_hbm, src_hbm, z_hbm, out_hbm, idx_v, e0, e1, s0, s1, acc):
        cid = lax.axis_index("c")
        sid = lax.axis_index("s")
        wid = sid * 2 + cid
        nch = per // 128
        pltpu.sync_copy(src_hbm.at[pl.ds(wid * nch, nch)], idx_v)
        pltpu.sync_copy(z_hbm.at[pl.ds(sid * stripe, stripe)],
                        acc.at[pl.ds(sid * stripe, stripe)])
        plsc.subcore_barrier()
        base = wid * per

        def start_l(c, buf, sem):
            pltpu.async_copy(ef_hbm.at[pl.ds(base + c * 128, 128)], buf, sem)

        def wait_l(buf, sem):
            pltpu.make_async_copy(ef_hbm.at[pl.ds(base, 128)], buf,
                                  sem).wait()

        start_l(0, e0, s0)

        @pl.loop(0, nch // 2)
        def _(hh):
            c0 = 2 * hh
            wait_l(e0, s0)
            start_l(c0 + 1, e1, s1)
            pltpu.sync_copy(e0, acc.at[idx_v.at[c0]], add=True)
            wait_l(e1, s1)

            @pl.when(hh < nch // 2 - 1)
            def _():
                start_l(c0 + 2, e0, s0)

            pltpu.sync_copy(e1, acc.at[idx_v.at[c0 + 1]], add=True)

        plsc.subcore_barrier()
        pltpu.sync_copy(acc.at[pl.ds(sid * stripe, stripe)],
                        out_hbm.at[cid].at[pl.ds(sid * stripe, stripe)])

    return sk(ef, srcp, zeros_np)


def _sc_counts(srcp, ones16, zeros16):
    stripe = NP // 16
    per = EP // NW

    @functools.partial(
        pl.kernel,
        out_type=jax.ShapeDtypeStruct((2, NP, 16), f32),
        mesh=_mesh(),
        scratch_types=[
            pltpu.VMEM((EP // NW // 128, 128), i32),
            pltpu.VMEM((128, 16), f32),
            pltpu.VMEM_SHARED((NP, 16), f32),
        ],
        compiler_params=pltpu.CompilerParams(use_tc_tiling_on_sc=False),
    )
    def ck(src_hbm, ones_hbm, z_hbm, out_hbm, idx_v, ones_v, acc):
        cid = lax.axis_index("c")
        sid = lax.axis_index("s")
        wid = sid * 2 + cid
        nch = per // 128
        pltpu.sync_copy(src_hbm.at[pl.ds(wid * nch, nch)], idx_v)
        pltpu.sync_copy(ones_hbm, ones_v)
        pltpu.sync_copy(z_hbm.at[pl.ds(sid * stripe, stripe)],
                        acc.at[pl.ds(sid * stripe, stripe)])
        plsc.subcore_barrier()

        @pl.loop(0, nch)
        def _(c):
            pltpu.sync_copy(ones_v, acc.at[idx_v.at[c]], add=True)

        plsc.subcore_barrier()
        pltpu.sync_copy(acc.at[pl.ds(sid * stripe, stripe)],
                        out_hbm.at[cid].at[pl.ds(sid * stripe, stripe)])

    return ck(srcp, ones16, zeros16)


def _sig(x):
    return 1.0 / (1.0 + jnp.exp(-x))


def _silu(x):
    return x * _sig(x)


def _ln(x, s, b):
    m = jnp.mean(x, axis=-1, keepdims=True)
    var = jnp.mean((x - m) ** 2, axis=-1, keepdims=True)
    return (x - m) * lax.rsqrt(var + 1e-5) * s + b


def _dot(a, b):
    return jnp.dot(a, b, preferred_element_type=f32,
                   precision=lax.Precision.HIGHEST)


def _full(a):
    return pl.BlockSpec(a.shape, lambda j: (0,) * a.ndim)


def _prologue(t, fw, emb128, ale_W, ale_b, h2, gid2, lmat, vpad1):
    def body(t_ref, fw_ref, emb_ref, aw_ref, ab_ref, h_ref, g_ref, lm_ref,
             vp_ref, nf_ref, lv_ref):
        xp = 2.0 * np.pi * _dot(t_ref[...], fw_ref[...])
        temb = jnp.concatenate([jnp.cos(xp), jnp.sin(xp)], axis=1)
        hh = h_ref[...]
        oh_h = (hh == lax.broadcasted_iota(i32, (NB, 128), 1)).astype(f32)
        nf_emb = _dot(oh_h, emb_ref[...])
        gg = g_ref[...]
        oh_g = (gg == lax.broadcasted_iota(i32, (NB, G), 1)).astype(f32)
        tpa = _dot(oh_g, temb)
        aw = aw_ref[...]
        nf = (_dot(nf_emb, aw[:128])
              + _dot(tpa, aw[128:])
              + ab_ref[...])
        nf_ref[...] = nf
        lv_ref[...] = _dot(oh_g, lm_ref[...]) + vp_ref[...]

    return pl.pallas_call(
        body,
        grid=(NP // NB,),
        in_specs=[
            _full(t), _full(fw), _full(emb128), _full(ale_W), _full(ale_b),
            pl.BlockSpec((NB, 1), lambda j: (j, 0)),
            pl.BlockSpec((NB, 1), lambda j: (j, 0)),
            _full(lmat),
            pl.BlockSpec((NB, 16), lambda j: (j, 0)),
        ],
        out_specs=[
            pl.BlockSpec((NB, D), lambda j: (j, 0)),
            pl.BlockSpec((NB, 16), lambda j: (j, 0)),
        ],
        out_shape=[
            jax.ShapeDtypeStruct((NP, D), f32),
            jax.ShapeDtypeStruct((NP, 16), f32),
        ],
    )(t, fw, emb128, ale_W, ale_b, h2, gid2, lmat, vpad1)


def _node_pre(nf, lv, lns, lnb, wh2, wnlv):
    def body(nf_ref, lv_ref, s_ref, b_ref, wh_ref, wl_ref, hf_ref, t2_ref):
        hf = _ln(nf_ref[...], s_ref[...], b_ref[...])
        hf_ref[...] = hf
        ab = (_dot(hf, wh_ref[...])
              + _dot(lv_ref[...], wl_ref[...]))
        t2_ref[0, :, :] = ab[:, :D]
        t2_ref[1, :, :] = ab[:, D:]

    return pl.pallas_call(
        body,
        grid=(NP // NB,),
        in_specs=[
            pl.BlockSpec((NB, D), lambda j: (j, 0)),
            pl.BlockSpec((NB, 16), lambda j: (j, 0)),
            _full(lns), _full(lnb), _full(wh2), _full(wnlv),
        ],
        out_specs=[
            pl.BlockSpec((NB, D), lambda j: (j, 0)),
            pl.BlockSpec((2, NB, D), lambda j: (0, j, 0)),
        ],
        out_shape=[
            jax.ShapeDtypeStruct((NP, D), f32),
            jax.ShapeDtypeStruct((2, NP, D), f32),
        ],
    )(nf, lv, lns, lnb, wh2, wnlv)


def _pd_prep(posg3, s16):
    def body(pg_ref, s_ref, out_ref):
        dvec = pg_ref[1, :, :] - pg_ref[0, :, :]
        e = _dot(dvec, s_ref[...])
        col = lax.broadcasted_iota(i32, (EB, 64), 1)
        out_ref[...] = (jnp.where(col < 30, jnp.sin(e), 0.0)
                        + jnp.where((col >= 32) & (col < 62), jnp.cos(e), 0.0))

    return pl.pallas_call(
        body,
        grid=(EP // EB,),
        in_specs=[
            pl.BlockSpec((2, EB, 16), lambda j: (0, j, 0)),
            _full(s16),
        ],
        out_specs=pl.BlockSpec((EB, 64), lambda j: (j, 0)),
        out_shape=jax.ShapeDtypeStruct((EP, 64), f32),
    )(posg3, s16)


def _edge(gat3, pdemb, wpd, w2, b2):
    def body(g_ref, pd_ref, wpd_ref, w2_ref, b2_ref, ef_ref):
        pre = (g_ref[0, :, :] + g_ref[1, :, :]
               + _dot(pd_ref[...], wpd_ref[...]))
        e1 = _silu(pre)
        z = _dot(e1, w2_ref[...]) + b2_ref[...]
        ef_ref[...] = _silu(z)

    return pl.pallas_call(
        body,
        grid=(EP // EB,),
        in_specs=[
            pl.BlockSpec((2, EB, D), lambda j: (0, j, 0)),
            pl.BlockSpec((EB, 64), lambda j: (j, 0)),
            _full(wpd), _full(w2), _full(b2),
        ],
        out_specs=pl.BlockSpec((EB, D), lambda j: (j, 0)),
        out_shape=jax.ShapeDtypeStruct((EP, D), f32),
    )(gat3, pdemb, wpd, w2, b2)


def _node_post(aggp, cntp, hf, nf, w1h, w1a, b1, w2, b2):
    def body(ag_ref, c_ref, hf_ref, nf_ref, w1h_ref, w1a_ref, b1_ref,
             w2_ref, b2_ref, out_ref):
        c = c_ref[0, :, 0:1] + c_ref[1, :, 0:1]
        agg = (ag_ref[0, :, :] + ag_ref[1, :, :]) / jnp.maximum(c, 1.0)
        n1 = _silu(_dot(hf_ref[...], w1h_ref[...])
                   + _dot(agg, w1a_ref[...])
                   + b1_ref[...])
        n2 = _silu(_dot(n1, w2_ref[...])
                   + b2_ref[...])
        out_ref[...] = nf_ref[...] + n2

    return pl.pallas_call(
        body,
        grid=(NP // NB,),
        in_specs=[
            pl.BlockSpec((2, NB, D), lambda j: (0, j, 0)),
            pl.BlockSpec((2, NB, 16), lambda j: (0, j, 0)),
            pl.BlockSpec((NB, D), lambda j: (j, 0)),
            pl.BlockSpec((NB, D), lambda j: (j, 0)),
            _full(w1h), _full(w1a), _full(b1), _full(w2), _full(b2),
        ],
        out_specs=pl.BlockSpec((NB, D), lambda j: (j, 0)),
        out_shape=jax.ShapeDtypeStruct((NP, D), f32),
    )(aggp, cntp, hf, nf, w1h, w1a, b1, w2, b2)


def _epi1(nf, gid2, fs, fb, ovw1, ovb1, ovw2p):
    def body(nf_ref, g_ref, fs_ref, fb_ref, w1_ref, b1_ref, w2_ref,
             ov_ref, sv_ref, gf_ref, cn_ref):
        nfn = _ln(nf_ref[...], fs_ref[...], fb_ref[...])
        ovh = _silu(_dot(nfn, w1_ref[...])
                    + b1_ref[...])
        ov = _dot(ovh, w2_ref[...])
        ov_ref[...] = ov
        oh = (g_ref[...] == lax.broadcasted_iota(i32, (NB, G), 1)).astype(f32)

        @pl.when(pl.program_id(0) == 0)
        def _():
            sv_ref[...] = jnp.zeros((G, D), f32)
            gf_ref[...] = jnp.zeros((G, D), f32)
            cn_ref[...] = jnp.zeros((G, D), f32)

        dn = (((0,), (0,)), ((), ()))
        sv_ref[...] += lax.dot_general(oh, ov, dn, preferred_element_type=f32, precision=lax.Precision.HIGHEST)
        gf_ref[...] += lax.dot_general(oh, nfn, dn, preferred_element_type=f32, precision=lax.Precision.HIGHEST)
        cn_ref[...] += lax.dot_general(oh, jnp.ones((NB, D), f32), dn,
                                       preferred_element_type=f32,
                                       precision=lax.Precision.HIGHEST)

    return pl.pallas_call(
        body,
        grid=(NP // NB,),
        in_specs=[
            pl.BlockSpec((NB, D), lambda j: (j, 0)),
            pl.BlockSpec((NB, 1), lambda j: (j, 0)),
            _full(fs), _full(fb), _full(ovw1), _full(ovb1), _full(ovw2p),
        ],
        out_specs=[
            pl.BlockSpec((NB, D), lambda j: (j, 0)),
            pl.BlockSpec((G, D), lambda j: (0, 0)),
            pl.BlockSpec((G, D), lambda j: (0, 0)),
            pl.BlockSpec((G, D), lambda j: (0, 0)),
        ],
        out_shape=[
            jax.ShapeDtypeStruct((NP, D), f32),
            jax.ShapeDtypeStruct((G, D), f32),
            jax.ShapeDtypeStruct((G, D), f32),
            jax.ShapeDtypeStruct((G, D), f32),
        ],
    )(nf, gid2, fs, fb, ovw1, ovb1, ovw2p)


def _epi2(sv, gf, cn, olwp):
    def body(sv_ref, gf_ref, cn_ref, ol_ref, mv_ref, gl_ref):
        c = jnp.maximum(cn_ref[...], 1.0)
        mv_ref[...] = sv_ref[...] / c
        gl_ref[...] = _dot(gf_ref[...] / c, ol_ref[...])

    return pl.pallas_call(
        body,
        grid=(1,),
        in_specs=[_full(sv), _full(gf), _full(cn), _full(olwp)],
        out_specs=[
            pl.BlockSpec((G, D), lambda j: (0, 0)),
            pl.BlockSpec((G, D), lambda j: (0, 0)),
        ],
        out_shape=[
            jax.ShapeDtypeStruct((G, D), f32),
            jax.ShapeDtypeStruct((G, D), f32),
        ],
    )(sv, gf, cn, olwp)


def _epi3(ov, gid2, mv):
    def body(ov_ref, g_ref, mv_ref, out_ref):
        oh = (g_ref[...] == lax.broadcasted_iota(i32, (NB, G), 1)).astype(f32)
        out_ref[...] = ov_ref[...] - _dot(oh, mv_ref[...])

    return pl.pallas_call(
        body,
        grid=(NP // NB,),
        in_specs=[
            pl.BlockSpec((NB, D), lambda j: (j, 0)),
            pl.BlockSpec((NB, 1), lambda j: (j, 0)),
            _full(mv),
        ],
        out_specs=pl.BlockSpec((NB, D), lambda j: (j, 0)),
        out_shape=jax.ShapeDtypeStruct((NP, D), f32),
    )(ov, gid2, mv)


def kernel(t, pos, v, l, emb_table, fourier_W, ale_W, ale_b, ln_s, ln_b, vW,
           vb, eW1, eb1, eW2, eb2, nW1, nb1, nW2, nb2, fln_s, fln_b, ovW1,
           ovb1, ovW2, olW, h, node_index, edge_node_index):
    src = edge_node_index[0].astype(i32)
    dst = edge_node_index[1].astype(i32)
    src_p = jnp.full((EP,), N, i32).at[:E].set(src)
    dst_p = jnp.zeros((EP,), i32).at[:E].set(dst)
    cat_idx = jnp.concatenate([src_p, dst_p + NP]).reshape(2 * EP // 128, 128)
    src_p2 = src_p.reshape(EP // 128, 128)

    gid2 = jnp.full((NP, 1), G, i32).at[:N, 0].set(node_index.astype(i32))
    h2 = jnp.zeros((NP, 1), i32).at[:N, 0].set(h.astype(i32))

    emb128 = jnp.zeros((128, D), f32).at[:101].set(emb_table)
    vpad1 = (jnp.zeros((NP, 16), f32).at[:N, 8:11].set(v)
             .at[:, 15].set(1.0))
    lmat = jnp.zeros((G, 16), f32).at[:, 0:6].set(l)
    pos16 = jnp.zeros((NP, 16), f32).at[:N, 0:3].set(pos)
    pos2 = jnp.concatenate([pos16, pos16], axis=0)

    zeros_np = jnp.zeros((NP, D), f32)
    zeros16 = jnp.zeros((NP, 16), f32)
    ones16 = jnp.ones((128, 16), f32)

    freqs = 2.0 * np.pi * np.arange(NFREQ, dtype=np.float32)
    s_np = np.zeros((16, 64), np.float32)
    for c in range(3):
        s_np[c, c * NFREQ:(c + 1) * NFREQ] = freqs
        s_np[c, 32 + c * NFREQ:32 + (c + 1) * NFREQ] = freqs
    s16 = jnp.asarray(s_np)

    wh2_l, wnlv_l, wpd_l = [], [], []
    for i in range(4):
        w_hi = eW1[i, 0:128]
        w_hj = eW1[i, 128:256]
        w_l = eW1[i, 256:262]
        w_v = eW1[i, 262:322]
        w_pd = eW1[i, 322:382]
        mv = vW[i] @ w_v
        bias_a = eb1[i] + vb[i] @ w_v
        wh2_l.append(jnp.concatenate([w_hi, w_hj], axis=1))
        wnlv = jnp.zeros((16, 2 * D), f32)
        wnlv = wnlv.at[0:6, :D].set(w_l)
        wnlv = wnlv.at[8:11, :D].set(-mv).at[8:11, D:].set(mv)
        wnlv = wnlv.at[15, :D].set(bias_a)
        wnlv_l.append(wnlv)
        wpd = jnp.zeros((64, D), f32)
        wpd = wpd.at[0:30].set(w_pd[0:30]).at[32:62].set(w_pd[30:60])
        wpd_l.append(wpd)

    ovw2p = jnp.zeros((D, D), f32).at[:, 0:3].set(ovW2)
    olwp = jnp.zeros((D, D), f32).at[:, 0:6].set(olW)

    nf, lv = _prologue(t, fourier_W, emb128, ale_W, ale_b, h2, gid2, lmat,
                       vpad1)
    posg = _sc_gather(pos2, cat_idx, 16).reshape(2, EP, 16)
    pdemb = _pd_prep(posg, s16)
    cntp = _sc_counts(src_p2, ones16, zeros16)

    for i in range(4):
        hf, t2 = _node_pre(nf, lv, ln_s[i].reshape(1, D),
                           ln_b[i].reshape(1, D), wh2_l[i], wnlv_l[i])
        gat = _sc_gather(t2.reshape(2 * NP, D), cat_idx, D).reshape(2, EP, D)
        ef = _edge(gat, pdemb, wpd_l[i], eW2[i], eb2[i].reshape(1, D))
        aggp = _sc_scatter_add(ef, src_p2, zeros_np)
        nf = _node_post(aggp, cntp, hf, nf, nW1[i, :D], nW1[i, D:],
                        nb1[i].reshape(1, D), nW2[i], nb2[i].reshape(1, D))

    ov, sv, gf, cn = _epi1(nf, gid2, fln_s.reshape(1, D),
                           fln_b.reshape(1, D), ovW1, ovb1.reshape(1, D),
                           ovw2p)
    mv_g, gl = _epi2(sv, gf, cn, olwp)
    ovc = _epi3(ov, gid2, mv_g)
    return ovc[:N, 0:3], gl[:, 0:6]


# speedup vs baseline: 1.1603x; 1.1603x over previous
import functools

import jax
import jax.numpy as jnp
import numpy as np
from jax import lax
from jax.experimental import pallas as pl
from jax.experimental.pallas import tpu as pltpu
from jax.experimental.pallas import tpu_sc as plsc

N = 10000
NP = 10240
E = 320000
EP = 327680
G = 64
D = 128
NW = 32
NB = 1280
EB = 1024
NFREQ = 10

f32 = jnp.float32
i32 = jnp.int32

def _mesh():
    return plsc.VectorSubcoreMesh(core_axis_name="c", subcore_axis_name="s")


def _sc_gather(table, idx2d, d, dt=f32):
    mrows = idx2d.shape[0]
    m = mrows * 128
    nch = mrows // NW
    nbuf = 4
    nh = nch // nbuf
    per = nch * 128

    @functools.partial(
        pl.kernel,
        out_type=jax.ShapeDtypeStruct((m, d), dt),
        mesh=_mesh(),
        scratch_types=[
            pltpu.VMEM((nch, 128), i32),
        ] + [pltpu.VMEM((128, d), dt)] * nbuf
          + [pltpu.SemaphoreType.DMA] * (2 * nbuf),
        compiler_params=pltpu.CompilerParams(use_tc_tiling_on_sc=(d == D)),
    )
    def gk(table_hbm, idx_hbm, out_hbm, idx_v, *bufsem):
        bufs = bufsem[:nbuf]
        gsems = bufsem[nbuf:2 * nbuf]
        osems = bufsem[2 * nbuf:]
        wid = lax.axis_index("s") * 2 + lax.axis_index("c")
        base = wid * per
        pltpu.sync_copy(idx_hbm.at[pl.ds(wid * nch, nch)], idx_v)

        def start_g(c, k):
            pltpu.async_copy(table_hbm.at[idx_v.at[c]], bufs[k], gsems[k])

        def wait_g(k):
            pltpu.make_async_copy(table_hbm.at[idx_v.at[0]], bufs[k],
                                  gsems[k]).wait()

        def start_o(c, k):
            pltpu.async_copy(bufs[k],
                             out_hbm.at[pl.ds(base + c * 128, 128)], osems[k])

        def wait_o(k):
            pltpu.make_async_copy(bufs[k], out_hbm.at[pl.ds(base, 128)],
                                  osems[k]).wait()

        for k in range(nbuf):
            start_g(k, k)

        @pl.loop(0, nh)
        def _(hh):
            c0 = nbuf * hh
            for k in range(nbuf):
                wait_g(k)
                start_o(c0 + k, k)

            @pl.when(hh < nh - 1)
            def _():
                for k in range(nbuf):
                    wait_o(k)
                    start_g(c0 + nbuf + k, k)

            @pl.when(hh == nh - 1)
            def _():
                for k in range(nbuf):
                    wait_o(k)

    return gk(table, idx2d)


def _sc_scatter_add(ef, srcp, zeros_np):
    stripe = NP // 16
    per = EP // NW

    @functools.partial(
        pl.kernel,
        out_type=jax.ShapeDtypeStruct((2, NP, D), f32),
        mesh=_mesh(),
        scratch_types=[
            pltpu.VMEM((EP // NW // 128, 128), i32),
            pltpu.VMEM((128, D), f32),
            pltpu.VMEM((128, D), f32),
            pltpu.SemaphoreType.DMA,
            pltpu.SemaphoreType.DMA,
            pltpu.VMEM_SHARED((NP, D), f32),
        ],
    )
    def sk(ef_hbm, src_hbm, z_hbm, out_hbm, idx_v, e0, e1, s0, s1, acc):
        cid = lax.axis_index("c")
        sid = lax.axis_index("s")
        wid = sid * 2 + cid
        nch = per // 128
        pltpu.sync_copy(src_hbm.at[pl.ds(wid * nch, nch)], idx_v)
        pltpu.sync_copy(z_hbm.at[pl.ds(sid * stripe, stripe)],
                        acc.at[pl.ds(sid * stripe, stripe)])
        plsc.subcore_barrier()
        base = wid * per

        def start_l(c, buf, sem):
            pltpu.async_copy(ef_hbm.at[pl.ds(base + c * 128, 128)], buf, sem)

        def wait_l(buf, sem):
            pltpu.make_async_copy(ef_hbm.at[pl.ds(base, 128)], buf,
                                  sem).wait()

        start_l(0, e0, s0)

        @pl.loop(0, nch // 2)
        def _(hh):
            c0 = 2 * hh
            wait_l(e0, s0)
            start_l(c0 + 1, e1, s1)
            pltpu.sync_copy(e0, acc.at[idx_v.at[c0]], add=True)
            wait_l(e1, s1)

            @pl.when(hh < nch // 2 - 1)
            def _():
                start_l(c0 + 2, e0, s0)

            pltpu.sync_copy(e1, acc.at[idx_v.at[c0 + 1]], add=True)

        plsc.subcore_barrier()
        pltpu.sync_copy(acc.at[pl.ds(sid * stripe, stripe)],
                        out_hbm.at[cid].at[pl.ds(sid * stripe, stripe)])

    return sk(ef, srcp, zeros_np)


def _sc_counts(srcp, ones16, zeros16):
    stripe = NP // 16
    per = EP // NW

    @functools.partial(
        pl.kernel,
        out_type=jax.ShapeDtypeStruct((2, NP, 16), f32),
        mesh=_mesh(),
        scratch_types=[
            pltpu.VMEM((EP // NW // 128, 128), i32),
            pltpu.VMEM((128, 16), f32),
            pltpu.VMEM_SHARED((NP, 16), f32),
        ],
        compiler_params=pltpu.CompilerParams(use_tc_tiling_on_sc=False),
    )
    def ck(src_hbm, ones_hbm, z_hbm, out_hbm, idx_v, ones_v, acc):
        cid = lax.axis_index("c")
        sid = lax.axis_index("s")
        wid = sid * 2 + cid
        nch = per // 128
        pltpu.sync_copy(src_hbm.at[pl.ds(wid * nch, nch)], idx_v)
        pltpu.sync_copy(ones_hbm, ones_v)
        pltpu.sync_copy(z_hbm.at[pl.ds(sid * stripe, stripe)],
                        acc.at[pl.ds(sid * stripe, stripe)])
        plsc.subcore_barrier()

        @pl.loop(0, nch)
        def _(c):
            pltpu.sync_copy(ones_v, acc.at[idx_v.at[c]], add=True)

        plsc.subcore_barrier()
        pltpu.sync_copy(acc.at[pl.ds(sid * stripe, stripe)],
                        out_hbm.at[cid].at[pl.ds(sid * stripe, stripe)])

    return ck(srcp, ones16, zeros16)


def _sig(x):
    return 1.0 / (1.0 + jnp.exp(-x))


def _silu(x):
    return x * _sig(x)


def _ln(x, s, b):
    m = jnp.mean(x, axis=-1, keepdims=True)
    var = jnp.mean((x - m) ** 2, axis=-1, keepdims=True)
    return (x - m) * lax.rsqrt(var + 1e-5) * s + b


def _dot(a, b):
    return jnp.dot(a, b, preferred_element_type=f32,
                   precision=lax.Precision.HIGHEST)


def _full(a):
    return pl.BlockSpec(a.shape, lambda j: (0,) * a.ndim)


def _prologue(t, fw, emb128, ale_W, ale_b, h2, gid2, lmat, vpad1):
    def body(t_ref, fw_ref, emb_ref, aw_ref, ab_ref, h_ref, g_ref, lm_ref,
             vp_ref, nf_ref, lv_ref):
        xp = 2.0 * np.pi * _dot(t_ref[...], fw_ref[...])
        temb = jnp.concatenate([jnp.cos(xp), jnp.sin(xp)], axis=1)
        hh = h_ref[...]
        oh_h = (hh == lax.broadcasted_iota(i32, (NB, 128), 1)).astype(f32)
        nf_emb = _dot(oh_h, emb_ref[...])
        gg = g_ref[...]
        oh_g = (gg == lax.broadcasted_iota(i32, (NB, G), 1)).astype(f32)
        tpa = _dot(oh_g, temb)
        aw = aw_ref[...]
        nf = (_dot(nf_emb, aw[:128])
              + _dot(tpa, aw[128:])
              + ab_ref[...])
        nf_ref[...] = nf
        lv_ref[...] = _dot(oh_g, lm_ref[...]) + vp_ref[...]

    return pl.pallas_call(
        body,
        grid=(NP // NB,),
        in_specs=[
            _full(t), _full(fw), _full(emb128), _full(ale_W), _full(ale_b),
            pl.BlockSpec((NB, 1), lambda j: (j, 0)),
            pl.BlockSpec((NB, 1), lambda j: (j, 0)),
            _full(lmat),
            pl.BlockSpec((NB, 16), lambda j: (j, 0)),
        ],
        out_specs=[
            pl.BlockSpec((NB, D), lambda j: (j, 0)),
            pl.BlockSpec((NB, 16), lambda j: (j, 0)),
        ],
        out_shape=[
            jax.ShapeDtypeStruct((NP, D), f32),
            jax.ShapeDtypeStruct((NP, 16), f32),
        ],
    )(t, fw, emb128, ale_W, ale_b, h2, gid2, lmat, vpad1)


def _node_pre(nf, lv, lns, lnb, wh2, wnlv):
    def body(nf_ref, lv_ref, s_ref, b_ref, wh_ref, wl_ref, hf_ref, t2_ref):
        hf = _ln(nf_ref[...], s_ref[...], b_ref[...])
        hf_ref[...] = hf
        ab = (_dot(hf, wh_ref[...])
              + _dot(lv_ref[...], wl_ref[...]))
        def pack(x):
            xi = lax.bitcast_convert_type(x, i32)
            xr = (xi + 0x7fff + ((xi >> 16) & 1)) >> 16
            return (xr[:, :64] << 16) | (xr[:, 64:] & 0xffff)

        t2_ref[0, :, :] = pack(ab[:, :D])
        t2_ref[1, :, :] = pack(ab[:, D:])

    return pl.pallas_call(
        body,
        grid=(NP // NB,),
        in_specs=[
            pl.BlockSpec((NB, D), lambda j: (j, 0)),
            pl.BlockSpec((NB, 16), lambda j: (j, 0)),
            _full(lns), _full(lnb), _full(wh2), _full(wnlv),
        ],
        out_specs=[
            pl.BlockSpec((NB, D), lambda j: (j, 0)),
            pl.BlockSpec((2, NB, D // 2), lambda j: (0, j, 0)),
        ],
        out_shape=[
            jax.ShapeDtypeStruct((NP, D), f32),
            jax.ShapeDtypeStruct((2, NP, D // 2), i32),
        ],
    )(nf, lv, lns, lnb, wh2, wnlv)


def _pd_prep(posg3, s16):
    def body(pg_ref, s_ref, out_ref):
        dvec = pg_ref[1, :, :] - pg_ref[0, :, :]
        e = _dot(dvec, s_ref[...])
        col = lax.broadcasted_iota(i32, (EB, 64), 1)
        out_ref[...] = (jnp.where(col < 30, jnp.sin(e), 0.0)
                        + jnp.where((col >= 32) & (col < 62), jnp.cos(e), 0.0))

    return pl.pallas_call(
        body,
        grid=(EP // EB,),
        in_specs=[
            pl.BlockSpec((2, EB, 16), lambda j: (0, j, 0)),
            _full(s16),
        ],
        out_specs=pl.BlockSpec((EB, 64), lambda j: (j, 0)),
        out_shape=jax.ShapeDtypeStruct((EP, 64), f32),
    )(posg3, s16)


def _edge(gat3, pdemb, wpd, w2, b2):
    def body(g_ref, pd_ref, wpd_ref, w2_ref, b2_ref, ef_ref):
        def unpack(x):
            hi = lax.bitcast_convert_type(x & jnp.int32(-65536), f32)
            lo = lax.bitcast_convert_type(x << 16, f32)
            return jnp.concatenate([hi, lo], axis=1)

        ga = unpack(g_ref[0, :, :])
        gb = unpack(g_ref[1, :, :])
        pre = (ga + gb
               + jnp.dot(pd_ref[...], wpd_ref[...],
                         preferred_element_type=f32))
        e1 = _silu(pre)
        z = jnp.dot(e1, w2_ref[...], preferred_element_type=f32) + b2_ref[...]
        ef_ref[...] = _silu(z)

    return pl.pallas_call(
        body,
        grid=(EP // EB,),
        in_specs=[
            pl.BlockSpec((2, EB, D // 2), lambda j: (0, j, 0)),
            pl.BlockSpec((EB, 64), lambda j: (j, 0)),
            _full(wpd), _full(w2), _full(b2),
        ],
        out_specs=pl.BlockSpec((EB, D), lambda j: (j, 0)),
        out_shape=jax.ShapeDtypeStruct((EP, D), f32),
    )(gat3, pdemb, wpd, w2, b2)


def _node_post(aggp, cntp, hf, nf, w1h, w1a, b1, w2, b2):
    def body(ag_ref, c_ref, hf_ref, nf_ref, w1h_ref, w1a_ref, b1_ref,
             w2_ref, b2_ref, out_ref):
        c = c_ref[0, :, 0:1] + c_ref[1, :, 0:1]
        agg = (ag_ref[0, :, :] + ag_ref[1, :, :]) / jnp.maximum(c, 1.0)
        n1 = _silu(_dot(hf_ref[...], w1h_ref[...])
                   + _dot(agg, w1a_ref[...])
                   + b1_ref[...])
        n2 = _silu(_dot(n1, w2_ref[...])
                   + b2_ref[...])
        out_ref[...] = nf_ref[...] + n2

    return pl.pallas_call(
        body,
        grid=(NP // NB,),
        in_specs=[
            pl.BlockSpec((2, NB, D), lambda j: (0, j, 0)),
            pl.BlockSpec((2, NB, 16), lambda j: (0, j, 0)),
            pl.BlockSpec((NB, D), lambda j: (j, 0)),
            pl.BlockSpec((NB, D), lambda j: (j, 0)),
            _full(w1h), _full(w1a), _full(b1), _full(w2), _full(b2),
        ],
        out_specs=pl.BlockSpec((NB, D), lambda j: (j, 0)),
        out_shape=jax.ShapeDtypeStruct((NP, D), f32),
    )(aggp, cntp, hf, nf, w1h, w1a, b1, w2, b2)


def _epi1(nf, gid2, fs, fb, ovw1, ovb1, ovw2p):
    def body(nf_ref, g_ref, fs_ref, fb_ref, w1_ref, b1_ref, w2_ref,
             ov_ref, sv_ref, gf_ref, cn_ref):
        nfn = _ln(nf_ref[...], fs_ref[...], fb_ref[...])
        ovh = _silu(_dot(nfn, w1_ref[...])
                    + b1_ref[...])
        ov = _dot(ovh, w2_ref[...])
        ov_ref[...] = ov
        oh = (g_ref[...] == lax.broadcasted_iota(i32, (NB, G), 1)).astype(f32)

        @pl.when(pl.program_id(0) == 0)
        def _():
            sv_ref[...] = jnp.zeros((G, D), f32)
            gf_ref[...] = jnp.zeros((G, D), f32)
            cn_ref[...] = jnp.zeros((G, D), f32)

        dn = (((0,), (0,)), ((), ()))
        sv_ref[...] += lax.dot_general(oh, ov, dn, preferred_element_type=f32, precision=lax.Precision.HIGHEST)
        gf_ref[...] += lax.dot_general(oh, nfn, dn, preferred_element_type=f32, precision=lax.Precision.HIGHEST)
        cn_ref[...] += lax.dot_general(oh, jnp.ones((NB, D), f32), dn,
                                       preferred_element_type=f32,
                                       precision=lax.Precision.HIGHEST)

    return pl.pallas_call(
        body,
        grid=(NP // NB,),
        in_specs=[
            pl.BlockSpec((NB, D), lambda j: (j, 0)),
            pl.BlockSpec((NB, 1), lambda j: (j, 0)),
            _full(fs), _full(fb), _full(ovw1), _full(ovb1), _full(ovw2p),
        ],
        out_specs=[
            pl.BlockSpec((NB, D), lambda j: (j, 0)),
            pl.BlockSpec((G, D), lambda j: (0, 0)),
            pl.BlockSpec((G, D), lambda j: (0, 0)),
            pl.BlockSpec((G, D), lambda j: (0, 0)),
        ],
        out_shape=[
            jax.ShapeDtypeStruct((NP, D), f32),
            jax.ShapeDtypeStruct((G, D), f32),
            jax.ShapeDtypeStruct((G, D), f32),
            jax.ShapeDtypeStruct((G, D), f32),
        ],
    )(nf, gid2, fs, fb, ovw1, ovb1, ovw2p)


def _epi2(sv, gf, cn, olwp):
    def body(sv_ref, gf_ref, cn_ref, ol_ref, mv_ref, gl_ref):
        c = jnp.maximum(cn_ref[...], 1.0)
        mv_ref[...] = sv_ref[...] / c
        gl_ref[...] = _dot(gf_ref[...] / c, ol_ref[...])

    return pl.pallas_call(
        body,
        grid=(1,),
        in_specs=[_full(sv), _full(gf), _full(cn), _full(olwp)],
        out_specs=[
            pl.BlockSpec((G, D), lambda j: (0, 0)),
            pl.BlockSpec((G, D), lambda j: (0, 0)),
        ],
        out_shape=[
            jax.ShapeDtypeStruct((G, D), f32),
            jax.ShapeDtypeStruct((G, D), f32),
        ],
    )(sv, gf, cn, olwp)


def _epi3(ov, gid2, mv):
    def body(ov_ref, g_ref, mv_ref, out_ref):
        oh = (g_ref[...] == lax.broadcasted_iota(i32, (NB, G), 1)).astype(f32)
        out_ref[...] = ov_ref[...] - _dot(oh, mv_ref[...])

    return pl.pallas_call(
        body,
        grid=(NP // NB,),
        in_specs=[
            pl.BlockSpec((NB, D), lambda j: (j, 0)),
            pl.BlockSpec((NB, 1), lambda j: (j, 0)),
            _full(mv),
        ],
        out_specs=pl.BlockSpec((NB, D), lambda j: (j, 0)),
        out_shape=jax.ShapeDtypeStruct((NP, D), f32),
    )(ov, gid2, mv)


def kernel(t, pos, v, l, emb_table, fourier_W, ale_W, ale_b, ln_s, ln_b, vW,
           vb, eW1, eb1, eW2, eb2, nW1, nb1, nW2, nb2, fln_s, fln_b, ovW1,
           ovb1, ovW2, olW, h, node_index, edge_node_index):
    src = edge_node_index[0].astype(i32)
    dst = edge_node_index[1].astype(i32)
    src_p = jnp.full((EP,), N, i32).at[:E].set(src)
    dst_p = jnp.zeros((EP,), i32).at[:E].set(dst)
    cat_idx = jnp.concatenate([src_p, dst_p + NP]).reshape(2 * EP // 128, 128)
    src_p2 = src_p.reshape(EP // 128, 128)

    gid2 = jnp.full((NP, 1), G, i32).at[:N, 0].set(node_index.astype(i32))
    h2 = jnp.zeros((NP, 1), i32).at[:N, 0].set(h.astype(i32))

    emb128 = jnp.zeros((128, D), f32).at[:101].set(emb_table)
    vpad1 = (jnp.zeros((NP, 16), f32).at[:N, 8:11].set(v)
             .at[:, 15].set(1.0))
    lmat = jnp.zeros((G, 16), f32).at[:, 0:6].set(l)
    pos16 = jnp.zeros((NP, 16), f32).at[:N, 0:3].set(pos)
    pos2 = jnp.concatenate([pos16, pos16], axis=0)

    zeros_np = jnp.zeros((NP, D), f32)
    zeros16 = jnp.zeros((NP, 16), f32)
    ones16 = jnp.ones((128, 16), f32)

    freqs = 2.0 * np.pi * np.arange(NFREQ, dtype=np.float32)
    s_np = np.zeros((16, 64), np.float32)
    for c in range(3):
        s_np[c, c * NFREQ:(c + 1) * NFREQ] = freqs
        s_np[c, 32 + c * NFREQ:32 + (c + 1) * NFREQ] = freqs
    s16 = jnp.asarray(s_np)

    wh2_l, wnlv_l, wpd_l = [], [], []
    for i in range(4):
        w_hi = eW1[i, 0:128]
        w_hj = eW1[i, 128:256]
        w_l = eW1[i, 256:262]
        w_v = eW1[i, 262:322]
        w_pd = eW1[i, 322:382]
        mv = vW[i] @ w_v
        bias_a = eb1[i] + vb[i] @ w_v
        wh2_l.append(jnp.concatenate([w_hi, w_hj], axis=1))
        wnlv = jnp.zeros((16, 2 * D), f32)
        wnlv = wnlv.at[0:6, :D].set(w_l)
        wnlv = wnlv.at[8:11, :D].set(-mv).at[8:11, D:].set(mv)
        wnlv = wnlv.at[15, :D].set(bias_a)
        wnlv_l.append(wnlv)
        wpd = jnp.zeros((64, D), f32)
        wpd = wpd.at[0:30].set(w_pd[0:30]).at[32:62].set(w_pd[30:60])
        wpd_l.append(wpd)

    ovw2p = jnp.zeros((D, D), f32).at[:, 0:3].set(ovW2)
    olwp = jnp.zeros((D, D), f32).at[:, 0:6].set(olW)

    nf, lv = _prologue(t, fourier_W, emb128, ale_W, ale_b, h2, gid2, lmat,
                       vpad1)
    posg = _sc_gather(pos2, cat_idx, 16).reshape(2, EP, 16)
    pdemb = _pd_prep(posg, s16)
    cntp = _sc_counts(src_p2, ones16, zeros16)

    for i in range(4):
        hf, t2 = _node_pre(nf, lv, ln_s[i].reshape(1, D),
                           ln_b[i].reshape(1, D), wh2_l[i], wnlv_l[i])
        gat = _sc_gather(t2.reshape(2 * NP, D // 2), cat_idx, D // 2,
                         i32).reshape(2, EP, D // 2)
        ef = _edge(gat, pdemb, wpd_l[i], eW2[i], eb2[i].reshape(1, D))
        aggp = _sc_scatter_add(ef, src_p2, zeros_np)
        nf = _node_post(aggp, cntp, hf, nf, nW1[i, :D], nW1[i, D:],
                        nb1[i].reshape(1, D), nW2[i], nb2[i].reshape(1, D))

    ov, sv, gf, cn = _epi1(nf, gid2, fln_s.reshape(1, D),
                           fln_b.reshape(1, D), ovW1, ovb1.reshape(1, D),
                           ovw2p)
    mv_g, gl = _epi2(sv, gf, cn, olwp)
    ovc = _epi3(ov, gid2, mv_g)
    return ovc[:N, 0:3], gl[:, 0:6]


# speedup vs baseline: 1.1985x; 1.0329x over previous
import functools

import jax
import jax.numpy as jnp
import numpy as np
from jax import lax
from jax.experimental import pallas as pl
from jax.experimental.pallas import tpu as pltpu
from jax.experimental.pallas import tpu_sc as plsc

N = 10000
NP = 10240
E = 320000
EP = 327680
G = 64
D = 128
NW = 32
NB = 1280
EB = 1024
NFREQ = 10

f32 = jnp.float32
i32 = jnp.int32

def _mesh():
    return plsc.VectorSubcoreMesh(core_axis_name="c", subcore_axis_name="s")


def _sc_gather(table, idx2d, d, dt=f32):
    mrows = idx2d.shape[0]
    m = mrows * 128
    nch = mrows // NW
    nbuf = 4
    nh = nch // nbuf
    per = nch * 128

    @functools.partial(
        pl.kernel,
        out_type=jax.ShapeDtypeStruct((m, d), dt),
        mesh=_mesh(),
        scratch_types=[
            pltpu.VMEM((nch, 128), i32),
        ] + [pltpu.VMEM((128, d), dt)] * nbuf
          + [pltpu.SemaphoreType.DMA] * (2 * nbuf),
        compiler_params=pltpu.CompilerParams(use_tc_tiling_on_sc=(d == D)),
    )
    def gk(table_hbm, idx_hbm, out_hbm, idx_v, *bufsem):
        bufs = bufsem[:nbuf]
        gsems = bufsem[nbuf:2 * nbuf]
        osems = bufsem[2 * nbuf:]
        wid = lax.axis_index("s") * 2 + lax.axis_index("c")
        base = wid * per
        pltpu.sync_copy(idx_hbm.at[pl.ds(wid * nch, nch)], idx_v)

        def start_g(c, k):
            pltpu.async_copy(table_hbm.at[idx_v.at[c]], bufs[k], gsems[k])

        def wait_g(k):
            pltpu.make_async_copy(table_hbm.at[idx_v.at[0]], bufs[k],
                                  gsems[k]).wait()

        def start_o(c, k):
            pltpu.async_copy(bufs[k],
                             out_hbm.at[pl.ds(base + c * 128, 128)], osems[k])

        def wait_o(k):
            pltpu.make_async_copy(bufs[k], out_hbm.at[pl.ds(base, 128)],
                                  osems[k]).wait()

        for k in range(nbuf):
            start_g(k, k)

        @pl.loop(0, nh)
        def _(hh):
            c0 = nbuf * hh
            for k in range(nbuf):
                wait_g(k)
                start_o(c0 + k, k)

            @pl.when(hh < nh - 1)
            def _():
                for k in range(nbuf):
                    wait_o(k)
                    start_g(c0 + nbuf + k, k)

            @pl.when(hh == nh - 1)
            def _():
                for k in range(nbuf):
                    wait_o(k)

    return gk(table, idx2d)


def _sc_scatter_add(ef, srcp, zeros_np):
    stripe = NP // 16
    per = EP // NW

    @functools.partial(
        pl.kernel,
        out_type=jax.ShapeDtypeStruct((2, NP, D), f32),
        mesh=_mesh(),
        scratch_types=[
            pltpu.VMEM((EP // NW // 128, 128), i32),
            pltpu.VMEM((128, D), f32),
            pltpu.VMEM((128, D), f32),
            pltpu.SemaphoreType.DMA,
            pltpu.SemaphoreType.DMA,
            pltpu.VMEM_SHARED((NP, D), f32),
        ],
    )
    def sk(ef_hbm, src_hbm, z_hbm, out_hbm, idx_v, e0, e1, s0, s1, acc):
        cid = lax.axis_index("c")
        sid = lax.axis_index("s")
        wid = sid * 2 + cid
        nch = per // 128
        pltpu.sync_copy(src_hbm.at[pl.ds(wid * nch, nch)], idx_v)
        pltpu.sync_copy(z_hbm.at[pl.ds(sid * stripe, stripe)],
                        acc.at[pl.ds(sid * stripe, stripe)])
        plsc.subcore_barrier()
        base = wid * per

        def start_l(c, buf, sem):
            pltpu.async_copy(ef_hbm.at[pl.ds(base + c * 128, 128)], buf, sem)

        def wait_l(buf, sem):
            pltpu.make_async_copy(ef_hbm.at[pl.ds(base, 128)], buf,
                                  sem).wait()

        start_l(0, e0, s0)

        @pl.loop(0, nch // 2)
        def _(hh):
            c0 = 2 * hh
            wait_l(e0, s0)
            start_l(c0 + 1, e1, s1)
            pltpu.sync_copy(e0, acc.at[idx_v.at[c0]], add=True)
            wait_l(e1, s1)

            @pl.when(hh < nch // 2 - 1)
            def _():
                start_l(c0 + 2, e0, s0)

            pltpu.sync_copy(e1, acc.at[idx_v.at[c0 + 1]], add=True)

        plsc.subcore_barrier()
        pltpu.sync_copy(acc.at[pl.ds(sid * stripe, stripe)],
                        out_hbm.at[cid].at[pl.ds(sid * stripe, stripe)])

    return sk(ef, srcp, zeros_np)


def _sc_counts(srcp, ones16, zeros16):
    stripe = NP // 16
    per = EP // NW

    @functools.partial(
        pl.kernel,
        out_type=jax.ShapeDtypeStruct((2, NP, 16), f32),
        mesh=_mesh(),
        scratch_types=[
            pltpu.VMEM((EP // NW // 128, 128), i32),
            pltpu.VMEM((128, 16), f32),
            pltpu.VMEM_SHARED((NP, 16), f32),
        ],
        compiler_params=pltpu.CompilerParams(use_tc_tiling_on_sc=False),
    )
    def ck(src_hbm, ones_hbm, z_hbm, out_hbm, idx_v, ones_v, acc):
        cid = lax.axis_index("c")
        sid = lax.axis_index("s")
        wid = sid * 2 + cid
        nch = per // 128
        pltpu.sync_copy(src_hbm.at[pl.ds(wid * nch, nch)], idx_v)
        pltpu.sync_copy(ones_hbm, ones_v)
        pltpu.sync_copy(z_hbm.at[pl.ds(sid * stripe, stripe)],
                        acc.at[pl.ds(sid * stripe, stripe)])
        plsc.subcore_barrier()

        @pl.loop(0, nch)
        def _(c):
            pltpu.sync_copy(ones_v, acc.at[idx_v.at[c]], add=True)

        plsc.subcore_barrier()
        pltpu.sync_copy(acc.at[pl.ds(sid * stripe, stripe)],
                        out_hbm.at[cid].at[pl.ds(sid * stripe, stripe)])

    return ck(srcp, ones16, zeros16)


def _sig(x):
    return 1.0 / (1.0 + jnp.exp(-x))


def _silu(x):
    return x * _sig(x)


def _ln(x, s, b):
    m = jnp.mean(x, axis=-1, keepdims=True)
    var = jnp.mean((x - m) ** 2, axis=-1, keepdims=True)
    return (x - m) * lax.rsqrt(var + 1e-5) * s + b


def _dot(a, b):
    return jnp.dot(a, b, preferred_element_type=f32,
                   precision=lax.Precision.HIGHEST)


def _full(a):
    return pl.BlockSpec(a.shape, lambda j: (0,) * a.ndim)


def _prologue(t, fw, emb128, ale_W, ale_b, h2, gid2, lmat, vpad1):
    def body(t_ref, fw_ref, emb_ref, aw_ref, ab_ref, h_ref, g_ref, lm_ref,
             vp_ref, nf_ref, lv_ref):
        xp = 2.0 * np.pi * _dot(t_ref[...], fw_ref[...])
        temb = jnp.concatenate([jnp.cos(xp), jnp.sin(xp)], axis=1)
        hh = h_ref[...]
        oh_h = (hh == lax.broadcasted_iota(i32, (NB, 128), 1)).astype(f32)
        nf_emb = _dot(oh_h, emb_ref[...])
        gg = g_ref[...]
        oh_g = (gg == lax.broadcasted_iota(i32, (NB, G), 1)).astype(f32)
        tpa = _dot(oh_g, temb)
        aw = aw_ref[...]
        nf = (_dot(nf_emb, aw[:128])
              + _dot(tpa, aw[128:])
              + ab_ref[...])
        nf_ref[...] = nf
        lv_ref[...] = _dot(oh_g, lm_ref[...]) + vp_ref[...]

    return pl.pallas_call(
        body,
        grid=(NP // NB,),
        in_specs=[
            _full(t), _full(fw), _full(emb128), _full(ale_W), _full(ale_b),
            pl.BlockSpec((NB, 1), lambda j: (j, 0)),
            pl.BlockSpec((NB, 1), lambda j: (j, 0)),
            _full(lmat),
            pl.BlockSpec((NB, 16), lambda j: (j, 0)),
        ],
        out_specs=[
            pl.BlockSpec((NB, D), lambda j: (j, 0)),
            pl.BlockSpec((NB, 16), lambda j: (j, 0)),
        ],
        out_shape=[
            jax.ShapeDtypeStruct((NP, D), f32),
            jax.ShapeDtypeStruct((NP, 16), f32),
        ],
    )(t, fw, emb128, ale_W, ale_b, h2, gid2, lmat, vpad1)


def _node_pre(nf, lv, lns, lnb, wh2, wnlv):
    def body(nf_ref, lv_ref, s_ref, b_ref, wh_ref, wl_ref, hf_ref, t2_ref):
        hf = _ln(nf_ref[...], s_ref[...], b_ref[...])
        hf_ref[...] = hf
        ab = (jnp.dot(hf, wh_ref[...], preferred_element_type=f32)
              + jnp.dot(lv_ref[...], wl_ref[...],
                        preferred_element_type=f32))
        def pack(x):
            xi = lax.bitcast_convert_type(x, i32)
            xr = (xi + 0x7fff + ((xi >> 16) & 1)) >> 16
            return (xr[:, :64] << 16) | (xr[:, 64:] & 0xffff)

        t2_ref[0, :, :] = pack(ab[:, :D])
        t2_ref[1, :, :] = pack(ab[:, D:])

    return pl.pallas_call(
        body,
        grid=(NP // NB,),
        in_specs=[
            pl.BlockSpec((NB, D), lambda j: (j, 0)),
            pl.BlockSpec((NB, 16), lambda j: (j, 0)),
            _full(lns), _full(lnb), _full(wh2), _full(wnlv),
        ],
        out_specs=[
            pl.BlockSpec((NB, D), lambda j: (j, 0)),
            pl.BlockSpec((2, NB, D // 2), lambda j: (0, j, 0)),
        ],
        out_shape=[
            jax.ShapeDtypeStruct((NP, D), f32),
            jax.ShapeDtypeStruct((2, NP, D // 2), i32),
        ],
    )(nf, lv, lns, lnb, wh2, wnlv)


def _pd_prep(posg3, s16):
    def body(pg_ref, s_ref, out_ref):
        dvec = pg_ref[1, :, :] - pg_ref[0, :, :]
        e = jnp.dot(dvec, s_ref[...], preferred_element_type=f32)
        col = lax.broadcasted_iota(i32, (EB, 64), 1)
        out_ref[...] = (jnp.where(col < 30, jnp.sin(e), 0.0)
                        + jnp.where((col >= 32) & (col < 62), jnp.cos(e), 0.0))

    return pl.pallas_call(
        body,
        grid=(EP // EB,),
        in_specs=[
            pl.BlockSpec((2, EB, 16), lambda j: (0, j, 0)),
            _full(s16),
        ],
        out_specs=pl.BlockSpec((EB, 64), lambda j: (j, 0)),
        out_shape=jax.ShapeDtypeStruct((EP, 64), f32),
    )(posg3, s16)


def _edge(gat3, pdemb, wpd, w2, b2):
    def body(g_ref, pd_ref, wpd_ref, w2_ref, b2_ref, ef_ref):
        def unpack(x):
            hi = lax.bitcast_convert_type(x & jnp.int32(-65536), f32)
            lo = lax.bitcast_convert_type(x << 16, f32)
            return jnp.concatenate([hi, lo], axis=1)

        ga = unpack(g_ref[0, :, :])
        gb = unpack(g_ref[1, :, :])
        pre = (ga + gb
               + jnp.dot(pd_ref[...], wpd_ref[...],
                         preferred_element_type=f32))
        e1 = _silu(pre)
        z = jnp.dot(e1, w2_ref[...], preferred_element_type=f32) + b2_ref[...]
        ef_ref[...] = _silu(z)

    return pl.pallas_call(
        body,
        grid=(EP // EB,),
        in_specs=[
            pl.BlockSpec((2, EB, D // 2), lambda j: (0, j, 0)),
            pl.BlockSpec((EB, 64), lambda j: (j, 0)),
            _full(wpd), _full(w2), _full(b2),
        ],
        out_specs=pl.BlockSpec((EB, D), lambda j: (j, 0)),
        out_shape=jax.ShapeDtypeStruct((EP, D), f32),
    )(gat3, pdemb, wpd, w2, b2)


def _node_post(aggp, cntp, hf, nf, w1h, w1a, b1, w2, b2):
    def body(ag_ref, c_ref, hf_ref, nf_ref, w1h_ref, w1a_ref, b1_ref,
             w2_ref, b2_ref, out_ref):
        c = c_ref[0, :, 0:1] + c_ref[1, :, 0:1]
        agg = (ag_ref[0, :, :] + ag_ref[1, :, :]) / jnp.maximum(c, 1.0)
        n1 = _silu(jnp.dot(hf_ref[...], w1h_ref[...],
                           preferred_element_type=f32)
                   + jnp.dot(agg, w1a_ref[...], preferred_element_type=f32)
                   + b1_ref[...])
        n2 = _silu(jnp.dot(n1, w2_ref[...], preferred_element_type=f32)
                   + b2_ref[...])
        out_ref[...] = nf_ref[...] + n2

    return pl.pallas_call(
        body,
        grid=(NP // NB,),
        in_specs=[
            pl.BlockSpec((2, NB, D), lambda j: (0, j, 0)),
            pl.BlockSpec((2, NB, 16), lambda j: (0, j, 0)),
            pl.BlockSpec((NB, D), lambda j: (j, 0)),
            pl.BlockSpec((NB, D), lambda j: (j, 0)),
            _full(w1h), _full(w1a), _full(b1), _full(w2), _full(b2),
        ],
        out_specs=pl.BlockSpec((NB, D), lambda j: (j, 0)),
        out_shape=jax.ShapeDtypeStruct((NP, D), f32),
    )(aggp, cntp, hf, nf, w1h, w1a, b1, w2, b2)


def _epi1(nf, gid2, fs, fb, ovw1, ovb1, ovw2p):
    def body(nf_ref, g_ref, fs_ref, fb_ref, w1_ref, b1_ref, w2_ref,
             ov_ref, sv_ref, gf_ref, cn_ref):
        nfn = _ln(nf_ref[...], fs_ref[...], fb_ref[...])
        ovh = _silu(_dot(nfn, w1_ref[...])
                    + b1_ref[...])
        ov = _dot(ovh, w2_ref[...])
        ov_ref[...] = ov
        oh = (g_ref[...] == lax.broadcasted_iota(i32, (NB, G), 1)).astype(f32)

        @pl.when(pl.program_id(0) == 0)
        def _():
            sv_ref[...] = jnp.zeros((G, D), f32)
            gf_ref[...] = jnp.zeros((G, D), f32)
            cn_ref[...] = jnp.zeros((G, D), f32)

        dn = (((0,), (0,)), ((), ()))
        sv_ref[...] += lax.dot_general(oh, ov, dn, preferred_element_type=f32, precision=lax.Precision.HIGHEST)
        gf_ref[...] += lax.dot_general(oh, nfn, dn, preferred_element_type=f32, precision=lax.Precision.HIGHEST)
        cn_ref[...] += lax.dot_general(oh, jnp.ones((NB, D), f32), dn,
                                       preferred_element_type=f32,
                                       precision=lax.Precision.HIGHEST)

    return pl.pallas_call(
        body,
        grid=(NP // NB,),
        in_specs=[
            pl.BlockSpec((NB, D), lambda j: (j, 0)),
            pl.BlockSpec((NB, 1), lambda j: (j, 0)),
            _full(fs), _full(fb), _full(ovw1), _full(ovb1), _full(ovw2p),
        ],
        out_specs=[
            pl.BlockSpec((NB, D), lambda j: (j, 0)),
            pl.BlockSpec((G, D), lambda j: (0, 0)),
            pl.BlockSpec((G, D), lambda j: (0, 0)),
            pl.BlockSpec((G, D), lambda j: (0, 0)),
        ],
        out_shape=[
            jax.ShapeDtypeStruct((NP, D), f32),
            jax.ShapeDtypeStruct((G, D), f32),
            jax.ShapeDtypeStruct((G, D), f32),
            jax.ShapeDtypeStruct((G, D), f32),
        ],
    )(nf, gid2, fs, fb, ovw1, ovb1, ovw2p)


def _epi2(sv, gf, cn, olwp):
    def body(sv_ref, gf_ref, cn_ref, ol_ref, mv_ref, gl_ref):
        c = jnp.maximum(cn_ref[...], 1.0)
        mv_ref[...] = sv_ref[...] / c
        gl_ref[...] = _dot(gf_ref[...] / c, ol_ref[...])

    return pl.pallas_call(
        body,
        grid=(1,),
        in_specs=[_full(sv), _full(gf), _full(cn), _full(olwp)],
        out_specs=[
            pl.BlockSpec((G, D), lambda j: (0, 0)),
            pl.BlockSpec((G, D), lambda j: (0, 0)),
        ],
        out_shape=[
            jax.ShapeDtypeStruct((G, D), f32),
            jax.ShapeDtypeStruct((G, D), f32),
        ],
    )(sv, gf, cn, olwp)


def _epi3(ov, gid2, mv):
    def body(ov_ref, g_ref, mv_ref, out_ref):
        oh = (g_ref[...] == lax.broadcasted_iota(i32, (NB, G), 1)).astype(f32)
        out_ref[...] = ov_ref[...] - _dot(oh, mv_ref[...])

    return pl.pallas_call(
        body,
        grid=(NP // NB,),
        in_specs=[
            pl.BlockSpec((NB, D), lambda j: (j, 0)),
            pl.BlockSpec((NB, 1), lambda j: (j, 0)),
            _full(mv),
        ],
        out_specs=pl.BlockSpec((NB, D), lambda j: (j, 0)),
        out_shape=jax.ShapeDtypeStruct((NP, D), f32),
    )(ov, gid2, mv)


def kernel(t, pos, v, l, emb_table, fourier_W, ale_W, ale_b, ln_s, ln_b, vW,
           vb, eW1, eb1, eW2, eb2, nW1, nb1, nW2, nb2, fln_s, fln_b, ovW1,
           ovb1, ovW2, olW, h, node_index, edge_node_index):
    src = edge_node_index[0].astype(i32)
    dst = edge_node_index[1].astype(i32)
    src_p = jnp.full((EP,), N, i32).at[:E].set(src)
    dst_p = jnp.zeros((EP,), i32).at[:E].set(dst)
    cat_idx = jnp.concatenate([src_p, dst_p + NP]).reshape(2 * EP // 128, 128)
    src_p2 = src_p.reshape(EP // 128, 128)

    gid2 = jnp.full((NP, 1), G, i32).at[:N, 0].set(node_index.astype(i32))
    h2 = jnp.zeros((NP, 1), i32).at[:N, 0].set(h.astype(i32))

    emb128 = jnp.zeros((128, D), f32).at[:101].set(emb_table)
    vpad1 = (jnp.zeros((NP, 16), f32).at[:N, 8:11].set(v)
             .at[:, 15].set(1.0))
    lmat = jnp.zeros((G, 16), f32).at[:, 0:6].set(l)
    pos16 = jnp.zeros((NP, 16), f32).at[:N, 0:3].set(pos)
    pos2 = jnp.concatenate([pos16, pos16], axis=0)

    zeros_np = jnp.zeros((NP, D), f32)
    zeros16 = jnp.zeros((NP, 16), f32)
    ones16 = jnp.ones((128, 16), f32)

    freqs = 2.0 * np.pi * np.arange(NFREQ, dtype=np.float32)
    s_np = np.zeros((16, 64), np.float32)
    for c in range(3):
        s_np[c, c * NFREQ:(c + 1) * NFREQ] = freqs
        s_np[c, 32 + c * NFREQ:32 + (c + 1) * NFREQ] = freqs
    s16 = jnp.asarray(s_np)

    wh2_l, wnlv_l, wpd_l = [], [], []
    for i in range(4):
        w_hi = eW1[i, 0:128]
        w_hj = eW1[i, 128:256]
        w_l = eW1[i, 256:262]
        w_v = eW1[i, 262:322]
        w_pd = eW1[i, 322:382]
        mv = vW[i] @ w_v
        bias_a = eb1[i] + vb[i] @ w_v
        wh2_l.append(jnp.concatenate([w_hi, w_hj], axis=1))
        wnlv = jnp.zeros((16, 2 * D), f32)
        wnlv = wnlv.at[0:6, :D].set(w_l)
        wnlv = wnlv.at[8:11, :D].set(-mv).at[8:11, D:].set(mv)
        wnlv = wnlv.at[15, :D].set(bias_a)
        wnlv_l.append(wnlv)
        wpd = jnp.zeros((64, D), f32)
        wpd = wpd.at[0:30].set(w_pd[0:30]).at[32:62].set(w_pd[30:60])
        wpd_l.append(wpd)

    ovw2p = jnp.zeros((D, D), f32).at[:, 0:3].set(ovW2)
    olwp = jnp.zeros((D, D), f32).at[:, 0:6].set(olW)

    nf, lv = _prologue(t, fourier_W, emb128, ale_W, ale_b, h2, gid2, lmat,
                       vpad1)
    posg = _sc_gather(pos2, cat_idx, 16).reshape(2, EP, 16)
    pdemb = _pd_prep(posg, s16)
    cntp = _sc_counts(src_p2, ones16, zeros16)

    for i in range(4):
        hf, t2 = _node_pre(nf, lv, ln_s[i].reshape(1, D),
                           ln_b[i].reshape(1, D), wh2_l[i], wnlv_l[i])
        gat = _sc_gather(t2.reshape(2 * NP, D // 2), cat_idx, D // 2,
                         i32).reshape(2, EP, D // 2)
        ef = _edge(gat, pdemb, wpd_l[i], eW2[i], eb2[i].reshape(1, D))
        aggp = _sc_scatter_add(ef, src_p2, zeros_np)
        nf = _node_post(aggp, cntp, hf, nf, nW1[i, :D], nW1[i, D:],
                        nb1[i].reshape(1, D), nW2[i], nb2[i].reshape(1, D))

    ov, sv, gf, cn = _epi1(nf, gid2, fln_s.reshape(1, D),
                           fln_b.reshape(1, D), ovW1, ovb1.reshape(1, D),
                           ovw2p)
    mv_g, gl = _epi2(sv, gf, cn, olwp)
    ovc = _epi3(ov, gid2, mv_g)
    return ovc[:N, 0:3], gl[:, 0:6]


# speedup vs baseline: 1.2553x; 1.0474x over previous
import functools

import jax
import jax.numpy as jnp
import numpy as np
from jax import lax
from jax.experimental import pallas as pl
from jax.experimental.pallas import tpu as pltpu
from jax.experimental.pallas import tpu_sc as plsc

N = 10000
NP = 10240
E = 320000
EP = 327680
G = 64
D = 128
NW = 32
NB = 1280
EB = 1024
NFREQ = 10

f32 = jnp.float32
i32 = jnp.int32

def _mesh():
    return plsc.VectorSubcoreMesh(core_axis_name="c", subcore_axis_name="s")


def _sc_gather(table, idx2d, d, dt=f32):
    mrows = idx2d.shape[0]
    m = mrows * 128
    nch = mrows // NW
    nbuf = 4
    nh = nch // nbuf
    per = nch * 128

    @functools.partial(
        pl.kernel,
        out_type=jax.ShapeDtypeStruct((m, d), dt),
        mesh=_mesh(),
        scratch_types=[
            pltpu.VMEM((nch, 128), i32),
        ] + [pltpu.VMEM((128, d), dt)] * nbuf
          + [pltpu.SemaphoreType.DMA] * (2 * nbuf),
        compiler_params=pltpu.CompilerParams(use_tc_tiling_on_sc=(d == D)),
    )
    def gk(table_hbm, idx_hbm, out_hbm, idx_v, *bufsem):
        bufs = bufsem[:nbuf]
        gsems = bufsem[nbuf:2 * nbuf]
        osems = bufsem[2 * nbuf:]
        wid = lax.axis_index("s") * 2 + lax.axis_index("c")
        base = wid * per
        pltpu.sync_copy(idx_hbm.at[pl.ds(wid * nch, nch)], idx_v)

        def start_g(c, k):
            pltpu.async_copy(table_hbm.at[idx_v.at[c]], bufs[k], gsems[k])

        def wait_g(k):
            pltpu.make_async_copy(table_hbm.at[idx_v.at[0]], bufs[k],
                                  gsems[k]).wait()

        def start_o(c, k):
            pltpu.async_copy(bufs[k],
                             out_hbm.at[pl.ds(base + c * 128, 128)], osems[k])

        def wait_o(k):
            pltpu.make_async_copy(bufs[k], out_hbm.at[pl.ds(base, 128)],
                                  osems[k]).wait()

        for k in range(nbuf):
            start_g(k, k)

        @pl.loop(0, nh)
        def _(hh):
            c0 = nbuf * hh
            for k in range(nbuf):
                wait_g(k)
                start_o(c0 + k, k)

            @pl.when(hh < nh - 1)
            def _():
                for k in range(nbuf):
                    wait_o(k)
                    start_g(c0 + nbuf + k, k)

            @pl.when(hh == nh - 1)
            def _():
                for k in range(nbuf):
                    wait_o(k)

    return gk(table, idx2d)


def _sc_scatter_add(ef, srcp, zeros_np):
    stripe = NP // 16
    per = ef.shape[0] // NW

    @functools.partial(
        pl.kernel,
        out_type=jax.ShapeDtypeStruct((2, NP, D), f32),
        mesh=_mesh(),
        scratch_types=[
            pltpu.VMEM((per // 128, 128), i32),
            pltpu.VMEM((128, D), f32),
            pltpu.VMEM((128, D), f32),
            pltpu.SemaphoreType.DMA,
            pltpu.SemaphoreType.DMA,
            pltpu.VMEM_SHARED((NP, D), f32),
        ],
    )
    def sk(ef_hbm, src_hbm, z_hbm, out_hbm, idx_v, e0, e1, s0, s1, acc):
        cid = lax.axis_index("c")
        sid = lax.axis_index("s")
        wid = sid * 2 + cid
        nch = per // 128
        pltpu.sync_copy(src_hbm.at[pl.ds(wid * nch, nch)], idx_v)
        pltpu.sync_copy(z_hbm.at[pl.ds(sid * stripe, stripe)],
                        acc.at[pl.ds(sid * stripe, stripe)])
        plsc.subcore_barrier()
        base = wid * per

        def start_l(c, buf, sem):
            pltpu.async_copy(ef_hbm.at[pl.ds(base + c * 128, 128)], buf, sem)

        def wait_l(buf, sem):
            pltpu.make_async_copy(ef_hbm.at[pl.ds(base, 128)], buf,
                                  sem).wait()

        start_l(0, e0, s0)

        @pl.loop(0, nch // 2)
        def _(hh):
            c0 = 2 * hh
            wait_l(e0, s0)
            start_l(c0 + 1, e1, s1)
            pltpu.sync_copy(e0, acc.at[idx_v.at[c0]], add=True)
            wait_l(e1, s1)

            @pl.when(hh < nch // 2 - 1)
            def _():
                start_l(c0 + 2, e0, s0)

            pltpu.sync_copy(e1, acc.at[idx_v.at[c0 + 1]], add=True)

        plsc.subcore_barrier()
        pltpu.sync_copy(acc.at[pl.ds(sid * stripe, stripe)],
                        out_hbm.at[cid].at[pl.ds(sid * stripe, stripe)])

    return sk(ef, srcp, zeros_np)


def _sc_counts(srcp, ones16, zeros16):
    stripe = NP // 16
    per = EP // NW

    @functools.partial(
        pl.kernel,
        out_type=jax.ShapeDtypeStruct((2, NP, 16), f32),
        mesh=_mesh(),
        scratch_types=[
            pltpu.VMEM((EP // NW // 128, 128), i32),
            pltpu.VMEM((128, 16), f32),
            pltpu.VMEM_SHARED((NP, 16), f32),
        ],
        compiler_params=pltpu.CompilerParams(use_tc_tiling_on_sc=False),
    )
    def ck(src_hbm, ones_hbm, z_hbm, out_hbm, idx_v, ones_v, acc):
        cid = lax.axis_index("c")
        sid = lax.axis_index("s")
        wid = sid * 2 + cid
        nch = per // 128
        pltpu.sync_copy(src_hbm.at[pl.ds(wid * nch, nch)], idx_v)
        pltpu.sync_copy(ones_hbm, ones_v)
        pltpu.sync_copy(z_hbm.at[pl.ds(sid * stripe, stripe)],
                        acc.at[pl.ds(sid * stripe, stripe)])
        plsc.subcore_barrier()

        @pl.loop(0, nch)
        def _(c):
            pltpu.sync_copy(ones_v, acc.at[idx_v.at[c]], add=True)

        plsc.subcore_barrier()
        pltpu.sync_copy(acc.at[pl.ds(sid * stripe, stripe)],
                        out_hbm.at[cid].at[pl.ds(sid * stripe, stripe)])

    return ck(srcp, ones16, zeros16)


def _sig(x):
    return 1.0 / (1.0 + jnp.exp(-x))


def _silu(x):
    return x * _sig(x)


def _ln(x, s, b):
    m = jnp.mean(x, axis=-1, keepdims=True)
    var = jnp.mean((x - m) ** 2, axis=-1, keepdims=True)
    return (x - m) * lax.rsqrt(var + 1e-5) * s + b


def _dot(a, b):
    return jnp.dot(a, b, preferred_element_type=f32,
                   precision=lax.Precision.HIGHEST)


def _full(a):
    return pl.BlockSpec(a.shape, lambda j: (0,) * a.ndim)


def _prologue(t, fw, emb128, ale_W, ale_b, h2, gid2, lmat, vpad1):
    def body(t_ref, fw_ref, emb_ref, aw_ref, ab_ref, h_ref, g_ref, lm_ref,
             vp_ref, nf_ref, lv_ref):
        xp = 2.0 * np.pi * _dot(t_ref[...], fw_ref[...])
        temb = jnp.concatenate([jnp.cos(xp), jnp.sin(xp)], axis=1)
        hh = h_ref[...]
        oh_h = (hh == lax.broadcasted_iota(i32, (NB, 128), 1)).astype(f32)
        nf_emb = _dot(oh_h, emb_ref[...])
        gg = g_ref[...]
        oh_g = (gg == lax.broadcasted_iota(i32, (NB, G), 1)).astype(f32)
        tpa = _dot(oh_g, temb)
        aw = aw_ref[...]
        nf = (_dot(nf_emb, aw[:128])
              + _dot(tpa, aw[128:])
              + ab_ref[...])
        nf_ref[...] = nf
        lv_ref[...] = _dot(oh_g, lm_ref[...]) + vp_ref[...]

    return pl.pallas_call(
        body,
        grid=(NP // NB,),
        in_specs=[
            _full(t), _full(fw), _full(emb128), _full(ale_W), _full(ale_b),
            pl.BlockSpec((NB, 1), lambda j: (j, 0)),
            pl.BlockSpec((NB, 1), lambda j: (j, 0)),
            _full(lmat),
            pl.BlockSpec((NB, 16), lambda j: (j, 0)),
        ],
        out_specs=[
            pl.BlockSpec((NB, D), lambda j: (j, 0)),
            pl.BlockSpec((NB, 16), lambda j: (j, 0)),
        ],
        out_shape=[
            jax.ShapeDtypeStruct((NP, D), f32),
            jax.ShapeDtypeStruct((NP, 16), f32),
        ],
    )(t, fw, emb128, ale_W, ale_b, h2, gid2, lmat, vpad1)


def _node_pre(nf, lv, lns, lnb, wh2, wnlv):
    def body(nf_ref, lv_ref, s_ref, b_ref, wh_ref, wl_ref, hf_ref, t2_ref):
        hf = _ln(nf_ref[...], s_ref[...], b_ref[...])
        hf_ref[...] = hf
        ab = (jnp.dot(hf, wh_ref[...], preferred_element_type=f32)
              + jnp.dot(lv_ref[...], wl_ref[...],
                        preferred_element_type=f32))
        def pack(x):
            xi = lax.bitcast_convert_type(x, i32)
            xr = (xi + 0x7fff + ((xi >> 16) & 1)) >> 16
            return (xr[:, :64] << 16) | (xr[:, 64:] & 0xffff)

        t2_ref[0, :, :] = pack(ab[:, :D])
        t2_ref[1, :, :] = pack(ab[:, D:])

    return pl.pallas_call(
        body,
        grid=(NP // NB,),
        in_specs=[
            pl.BlockSpec((NB, D), lambda j: (j, 0)),
            pl.BlockSpec((NB, 16), lambda j: (j, 0)),
            _full(lns), _full(lnb), _full(wh2), _full(wnlv),
        ],
        out_specs=[
            pl.BlockSpec((NB, D), lambda j: (j, 0)),
            pl.BlockSpec((2, NB, D // 2), lambda j: (0, j, 0)),
        ],
        out_shape=[
            jax.ShapeDtypeStruct((NP, D), f32),
            jax.ShapeDtypeStruct((2, NP, D // 2), i32),
        ],
    )(nf, lv, lns, lnb, wh2, wnlv)


def _pd_prep(posg3, s16):
    def body(pg_ref, s_ref, out_ref):
        dvec = pg_ref[1, :, :] - pg_ref[0, :, :]
        e = jnp.dot(dvec, s_ref[...], preferred_element_type=f32)
        col = lax.broadcasted_iota(i32, (EB, 64), 1)
        out_ref[...] = (jnp.where(col < 30, jnp.sin(e), 0.0)
                        + jnp.where((col >= 32) & (col < 62), jnp.cos(e), 0.0))

    return pl.pallas_call(
        body,
        grid=(EP // EB,),
        in_specs=[
            pl.BlockSpec((2, EB, 16), lambda j: (0, j, 0)),
            _full(s16),
        ],
        out_specs=pl.BlockSpec((EB, 64), lambda j: (j, 0)),
        out_shape=jax.ShapeDtypeStruct((EP, 64), f32),
    )(posg3, s16)


def _edge(gat3, pdemb, wpd, w2, b2, off):
    ec = gat3.shape[1]
    def body(g_ref, pd_ref, wpd_ref, w2_ref, b2_ref, ef_ref):
        def unpack(x):
            hi = lax.bitcast_convert_type(x & jnp.int32(-65536), f32)
            lo = lax.bitcast_convert_type(x << 16, f32)
            return jnp.concatenate([hi, lo], axis=1)

        ga = unpack(g_ref[0, :, :])
        gb = unpack(g_ref[1, :, :])
        pre = (ga + gb
               + jnp.dot(pd_ref[...], wpd_ref[...],
                         preferred_element_type=f32))
        e1 = _silu(pre)
        z = jnp.dot(e1, w2_ref[...], preferred_element_type=f32) + b2_ref[...]
        ef_ref[...] = _silu(z)

    return pl.pallas_call(
        body,
        grid=(ec // EB,),
        in_specs=[
            pl.BlockSpec((2, EB, D // 2), lambda j: (0, j, 0)),
            pl.BlockSpec((EB, 64), lambda j: (j + off, 0)),
            _full(wpd), _full(w2), _full(b2),
        ],
        out_specs=pl.BlockSpec((EB, D), lambda j: (j, 0)),
        out_shape=jax.ShapeDtypeStruct((ec, D), f32),
    )(gat3, pdemb, wpd, w2, b2)


def _node_post(aggp, aggq, cntp, hf, nf, w1h, w1a, b1, w2, b2):
    def body(ag_ref, aq_ref, c_ref, hf_ref, nf_ref, w1h_ref, w1a_ref, b1_ref,
             w2_ref, b2_ref, out_ref):
        c = c_ref[0, :, 0:1] + c_ref[1, :, 0:1]
        agg = ((ag_ref[0, :, :] + ag_ref[1, :, :])
               + (aq_ref[0, :, :] + aq_ref[1, :, :])) / jnp.maximum(c, 1.0)
        n1 = _silu(jnp.dot(hf_ref[...], w1h_ref[...],
                           preferred_element_type=f32)
                   + jnp.dot(agg, w1a_ref[...], preferred_element_type=f32)
                   + b1_ref[...])
        n2 = _silu(jnp.dot(n1, w2_ref[...], preferred_element_type=f32)
                   + b2_ref[...])
        out_ref[...] = nf_ref[...] + n2

    return pl.pallas_call(
        body,
        grid=(NP // NB,),
        in_specs=[
            pl.BlockSpec((2, NB, D), lambda j: (0, j, 0)),
            pl.BlockSpec((2, NB, D), lambda j: (0, j, 0)),
            pl.BlockSpec((2, NB, 16), lambda j: (0, j, 0)),
            pl.BlockSpec((NB, D), lambda j: (j, 0)),
            pl.BlockSpec((NB, D), lambda j: (j, 0)),
            _full(w1h), _full(w1a), _full(b1), _full(w2), _full(b2),
        ],
        out_specs=pl.BlockSpec((NB, D), lambda j: (j, 0)),
        out_shape=jax.ShapeDtypeStruct((NP, D), f32),
    )(aggp, aggq, cntp, hf, nf, w1h, w1a, b1, w2, b2)


def _epi1(nf, gid2, fs, fb, ovw1, ovb1, ovw2p):
    def body(nf_ref, g_ref, fs_ref, fb_ref, w1_ref, b1_ref, w2_ref,
             ov_ref, sv_ref, gf_ref, cn_ref):
        nfn = _ln(nf_ref[...], fs_ref[...], fb_ref[...])
        ovh = _silu(_dot(nfn, w1_ref[...])
                    + b1_ref[...])
        ov = _dot(ovh, w2_ref[...])
        ov_ref[...] = ov
        oh = (g_ref[...] == lax.broadcasted_iota(i32, (NB, G), 1)).astype(f32)

        @pl.when(pl.program_id(0) == 0)
        def _():
            sv_ref[...] = jnp.zeros((G, D), f32)
            gf_ref[...] = jnp.zeros((G, D), f32)
            cn_ref[...] = jnp.zeros((G, D), f32)

        dn = (((0,), (0,)), ((), ()))
        sv_ref[...] += lax.dot_general(oh, ov, dn, preferred_element_type=f32, precision=lax.Precision.HIGHEST)
        gf_ref[...] += lax.dot_general(oh, nfn, dn, preferred_element_type=f32, precision=lax.Precision.HIGHEST)
        cn_ref[...] += lax.dot_general(oh, jnp.ones((NB, D), f32), dn,
                                       preferred_element_type=f32,
                                       precision=lax.Precision.HIGHEST)

    return pl.pallas_call(
        body,
        grid=(NP // NB,),
        in_specs=[
            pl.BlockSpec((NB, D), lambda j: (j, 0)),
            pl.BlockSpec((NB, 1), lambda j: (j, 0)),
            _full(fs), _full(fb), _full(ovw1), _full(ovb1), _full(ovw2p),
        ],
        out_specs=[
            pl.BlockSpec((NB, D), lambda j: (j, 0)),
            pl.BlockSpec((G, D), lambda j: (0, 0)),
            pl.BlockSpec((G, D), lambda j: (0, 0)),
            pl.BlockSpec((G, D), lambda j: (0, 0)),
        ],
        out_shape=[
            jax.ShapeDtypeStruct((NP, D), f32),
            jax.ShapeDtypeStruct((G, D), f32),
            jax.ShapeDtypeStruct((G, D), f32),
            jax.ShapeDtypeStruct((G, D), f32),
        ],
    )(nf, gid2, fs, fb, ovw1, ovb1, ovw2p)


def _epi2(sv, gf, cn, olwp):
    def body(sv_ref, gf_ref, cn_ref, ol_ref, mv_ref, gl_ref):
        c = jnp.maximum(cn_ref[...], 1.0)
        mv_ref[...] = sv_ref[...] / c
        gl_ref[...] = _dot(gf_ref[...] / c, ol_ref[...])

    return pl.pallas_call(
        body,
        grid=(1,),
        in_specs=[_full(sv), _full(gf), _full(cn), _full(olwp)],
        out_specs=[
            pl.BlockSpec((G, D), lambda j: (0, 0)),
            pl.BlockSpec((G, D), lambda j: (0, 0)),
        ],
        out_shape=[
            jax.ShapeDtypeStruct((G, D), f32),
            jax.ShapeDtypeStruct((G, D), f32),
        ],
    )(sv, gf, cn, olwp)


def _epi3(ov, gid2, mv):
    def body(ov_ref, g_ref, mv_ref, out_ref):
        oh = (g_ref[...] == lax.broadcasted_iota(i32, (NB, G), 1)).astype(f32)
        out_ref[...] = ov_ref[...] - _dot(oh, mv_ref[...])

    return pl.pallas_call(
        body,
        grid=(NP // NB,),
        in_specs=[
            pl.BlockSpec((NB, D), lambda j: (j, 0)),
            pl.BlockSpec((NB, 1), lambda j: (j, 0)),
            _full(mv),
        ],
        out_specs=pl.BlockSpec((NB, D), lambda j: (j, 0)),
        out_shape=jax.ShapeDtypeStruct((NP, D), f32),
    )(ov, gid2, mv)


def kernel(t, pos, v, l, emb_table, fourier_W, ale_W, ale_b, ln_s, ln_b, vW,
           vb, eW1, eb1, eW2, eb2, nW1, nb1, nW2, nb2, fln_s, fln_b, ovW1,
           ovb1, ovW2, olW, h, node_index, edge_node_index):
    src = edge_node_index[0].astype(i32)
    dst = edge_node_index[1].astype(i32)
    src_p = jnp.full((EP,), N, i32).at[:E].set(src)
    dst_p = jnp.zeros((EP,), i32).at[:E].set(dst)
    cat_idx = jnp.concatenate([src_p, dst_p + NP]).reshape(2 * EP // 128, 128)
    src_p2 = src_p.reshape(EP // 128, 128)

    EPH = EP // 2
    cat_idx_c = [
        jnp.concatenate([src_p[c * EPH:(c + 1) * EPH],
                         dst_p[c * EPH:(c + 1) * EPH] + NP])
        .reshape(2 * EPH // 128, 128)
        for c in range(2)
    ]
    src_p2_c = [src_p[c * EPH:(c + 1) * EPH].reshape(EPH // 128, 128)
                for c in range(2)]

    gid2 = jnp.full((NP, 1), G, i32).at[:N, 0].set(node_index.astype(i32))
    h2 = jnp.zeros((NP, 1), i32).at[:N, 0].set(h.astype(i32))

    emb128 = jnp.zeros((128, D), f32).at[:101].set(emb_table)
    vpad1 = (jnp.zeros((NP, 16), f32).at[:N, 8:11].set(v)
             .at[:, 15].set(1.0))
    lmat = jnp.zeros((G, 16), f32).at[:, 0:6].set(l)
    pos16 = jnp.zeros((NP, 16), f32).at[:N, 0:3].set(pos)
    pos2 = jnp.concatenate([pos16, pos16], axis=0)

    zeros_np = jnp.zeros((NP, D), f32)
    zeros16 = jnp.zeros((NP, 16), f32)
    ones16 = jnp.ones((128, 16), f32)

    freqs = 2.0 * np.pi * np.arange(NFREQ, dtype=np.float32)
    s_np = np.zeros((16, 64), np.float32)
    for c in range(3):
        s_np[c, c * NFREQ:(c + 1) * NFREQ] = freqs
        s_np[c, 32 + c * NFREQ:32 + (c + 1) * NFREQ] = freqs
    s16 = jnp.asarray(s_np)

    wh2_l, wnlv_l, wpd_l = [], [], []
    for i in range(4):
        w_hi = eW1[i, 0:128]
        w_hj = eW1[i, 128:256]
        w_l = eW1[i, 256:262]
        w_v = eW1[i, 262:322]
        w_pd = eW1[i, 322:382]
        mv = vW[i] @ w_v
        bias_a = eb1[i] + vb[i] @ w_v
        wh2_l.append(jnp.concatenate([w_hi, w_hj], axis=1))
        wnlv = jnp.zeros((16, 2 * D), f32)
        wnlv = wnlv.at[0:6, :D].set(w_l)
        wnlv = wnlv.at[8:11, :D].set(-mv).at[8:11, D:].set(mv)
        wnlv = wnlv.at[15, :D].set(bias_a)
        wnlv_l.append(wnlv)
        wpd = jnp.zeros((64, D), f32)
        wpd = wpd.at[0:30].set(w_pd[0:30]).at[32:62].set(w_pd[30:60])
        wpd_l.append(wpd)

    ovw2p = jnp.zeros((D, D), f32).at[:, 0:3].set(ovW2)
    olwp = jnp.zeros((D, D), f32).at[:, 0:6].set(olW)

    nf, lv = _prologue(t, fourier_W, emb128, ale_W, ale_b, h2, gid2, lmat,
                       vpad1)
    posg = _sc_gather(pos2, cat_idx, 16).reshape(2, EP, 16)
    pdemb = _pd_prep(posg, s16)
    cntp = _sc_counts(src_p2, ones16, zeros16)

    for i in range(4):
        hf, t2 = _node_pre(nf, lv, ln_s[i].reshape(1, D),
                           ln_b[i].reshape(1, D), wh2_l[i], wnlv_l[i])
        t2r = t2.reshape(2 * NP, D // 2)
        gat0 = _sc_gather(t2r, cat_idx_c[0], D // 2,
                          i32).reshape(2, EPH, D // 2)
        ef0 = _edge(gat0, pdemb, wpd_l[i], eW2[i], eb2[i].reshape(1, D), 0)
        gat1 = _sc_gather(t2r, cat_idx_c[1], D // 2,
                          i32).reshape(2, EPH, D // 2)
        agg0 = _sc_scatter_add(ef0, src_p2_c[0], zeros_np)
        ef1 = _edge(gat1, pdemb, wpd_l[i], eW2[i], eb2[i].reshape(1, D),
                    EPH // EB)
        agg1 = _sc_scatter_add(ef1, src_p2_c[1], zeros_np)
        nf = _node_post(agg0, agg1, cntp, hf, nf, nW1[i, :D], nW1[i, D:],
                        nb1[i].reshape(1, D), nW2[i], nb2[i].reshape(1, D))

    ov, sv, gf, cn = _epi1(nf, gid2, fln_s.reshape(1, D),
                           fln_b.reshape(1, D), ovW1, ovb1.reshape(1, D),
                           ovw2p)
    mv_g, gl = _epi2(sv, gf, cn, olwp)
    ovc = _epi3(ov, gid2, mv_g)
    return ovc[:N, 0:3], gl[:, 0:6]


# speedup vs baseline: 1.3507x; 1.0760x over previous
import functools

import jax
import jax.numpy as jnp
import numpy as np
from jax import lax
from jax.experimental import pallas as pl
from jax.experimental.pallas import tpu as pltpu
from jax.experimental.pallas import tpu_sc as plsc

N = 10000
NP = 10240
E = 320000
EP = 327680
G = 64
D = 128
NW = 32
NB = 1280
EB = 1024
NFREQ = 10

f32 = jnp.float32
i32 = jnp.int32

def _mesh():
    return plsc.VectorSubcoreMesh(core_axis_name="c", subcore_axis_name="s")


def _sc_gather(table, idx2d, d, dt=f32):
    mrows = idx2d.shape[0]
    m = mrows * 128
    nch = mrows // NW
    nbuf = 4
    nh = nch // nbuf
    per = nch * 128

    @functools.partial(
        pl.kernel,
        out_type=jax.ShapeDtypeStruct((m, d), dt),
        mesh=_mesh(),
        scratch_types=[
            pltpu.VMEM((nch, 128), i32),
        ] + [pltpu.VMEM((128, d), dt)] * nbuf
          + [pltpu.SemaphoreType.DMA] * (2 * nbuf),
        compiler_params=pltpu.CompilerParams(use_tc_tiling_on_sc=(d == D)),
    )
    def gk(table_hbm, idx_hbm, out_hbm, idx_v, *bufsem):
        bufs = bufsem[:nbuf]
        gsems = bufsem[nbuf:2 * nbuf]
        osems = bufsem[2 * nbuf:]
        wid = lax.axis_index("s") * 2 + lax.axis_index("c")
        base = wid * per
        pltpu.sync_copy(idx_hbm.at[pl.ds(wid * nch, nch)], idx_v)

        def start_g(c, k):
            pltpu.async_copy(table_hbm.at[idx_v.at[c]], bufs[k], gsems[k])

        def wait_g(k):
            pltpu.make_async_copy(table_hbm.at[idx_v.at[0]], bufs[k],
                                  gsems[k]).wait()

        def start_o(c, k):
            pltpu.async_copy(bufs[k],
                             out_hbm.at[pl.ds(base + c * 128, 128)], osems[k])

        def wait_o(k):
            pltpu.make_async_copy(bufs[k], out_hbm.at[pl.ds(base, 128)],
                                  osems[k]).wait()

        for k in range(nbuf):
            start_g(k, k)

        @pl.loop(0, nh)
        def _(hh):
            c0 = nbuf * hh
            for k in range(nbuf):
                wait_g(k)
                start_o(c0 + k, k)

            @pl.when(hh < nh - 1)
            def _():
                for k in range(nbuf):
                    wait_o(k)
                    start_g(c0 + nbuf + k, k)

            @pl.when(hh == nh - 1)
            def _():
                for k in range(nbuf):
                    wait_o(k)

    return gk(table, idx2d)


def _sc_scatter_add(ef, srcp, zeros_np):
    stripe = NP // 16
    per = ef.shape[0] // NW

    @functools.partial(
        pl.kernel,
        out_type=jax.ShapeDtypeStruct((2, NP, D), f32),
        mesh=_mesh(),
        scratch_types=[
            pltpu.VMEM((per // 128, 128), i32),
            pltpu.VMEM((128, D), f32),
            pltpu.VMEM((128, D), f32),
            pltpu.SemaphoreType.DMA,
            pltpu.SemaphoreType.DMA,
            pltpu.VMEM_SHARED((NP, D), f32),
        ],
    )
    def sk(ef_hbm, src_hbm, z_hbm, out_hbm, idx_v, e0, e1, s0, s1, acc):
        cid = lax.axis_index("c")
        sid = lax.axis_index("s")
        wid = sid * 2 + cid
        nch = per // 128
        pltpu.sync_copy(src_hbm.at[pl.ds(wid * nch, nch)], idx_v)
        pltpu.sync_copy(z_hbm.at[pl.ds(sid * stripe, stripe)],
                        acc.at[pl.ds(sid * stripe, stripe)])
        plsc.subcore_barrier()
        base = wid * per

        def start_l(c, buf, sem):
            pltpu.async_copy(ef_hbm.at[pl.ds(base + c * 128, 128)], buf, sem)

        def wait_l(buf, sem):
            pltpu.make_async_copy(ef_hbm.at[pl.ds(base, 128)], buf,
                                  sem).wait()

        start_l(0, e0, s0)

        @pl.loop(0, nch // 2)
        def _(hh):
            c0 = 2 * hh
            wait_l(e0, s0)
            start_l(c0 + 1, e1, s1)
            pltpu.sync_copy(e0, acc.at[idx_v.at[c0]], add=True)
            wait_l(e1, s1)

            @pl.when(hh < nch // 2 - 1)
            def _():
                start_l(c0 + 2, e0, s0)

            pltpu.sync_copy(e1, acc.at[idx_v.at[c0 + 1]], add=True)

        plsc.subcore_barrier()
        pltpu.sync_copy(acc.at[pl.ds(sid * stripe, stripe)],
                        out_hbm.at[cid].at[pl.ds(sid * stripe, stripe)])

    return sk(ef, srcp, zeros_np)


def _sc_counts(srcp, ones16, zeros16):
    stripe = NP // 16
    per = EP // NW

    @functools.partial(
        pl.kernel,
        out_type=jax.ShapeDtypeStruct((2, NP, 16), f32),
        mesh=_mesh(),
        scratch_types=[
            pltpu.VMEM((EP // NW // 128, 128), i32),
            pltpu.VMEM((128, 16), f32),
            pltpu.VMEM_SHARED((NP, 16), f32),
        ],
        compiler_params=pltpu.CompilerParams(use_tc_tiling_on_sc=False),
    )
    def ck(src_hbm, ones_hbm, z_hbm, out_hbm, idx_v, ones_v, acc):
        cid = lax.axis_index("c")
        sid = lax.axis_index("s")
        wid = sid * 2 + cid
        nch = per // 128
        pltpu.sync_copy(src_hbm.at[pl.ds(wid * nch, nch)], idx_v)
        pltpu.sync_copy(ones_hbm, ones_v)
        pltpu.sync_copy(z_hbm.at[pl.ds(sid * stripe, stripe)],
                        acc.at[pl.ds(sid * stripe, stripe)])
        plsc.subcore_barrier()

        @pl.loop(0, nch)
        def _(c):
            pltpu.sync_copy(ones_v, acc.at[idx_v.at[c]], add=True)

        plsc.subcore_barrier()
        pltpu.sync_copy(acc.at[pl.ds(sid * stripe, stripe)],
                        out_hbm.at[cid].at[pl.ds(sid * stripe, stripe)])

    return ck(srcp, ones16, zeros16)


def _sig(x):
    return 1.0 / (1.0 + jnp.exp(-x))


def _silu(x):
    return x * _sig(x)


def _ln(x, s, b):
    m = jnp.mean(x, axis=-1, keepdims=True)
    var = jnp.mean((x - m) ** 2, axis=-1, keepdims=True)
    return (x - m) * lax.rsqrt(var + 1e-5) * s + b


def _dot(a, b):
    return jnp.dot(a, b, preferred_element_type=f32,
                   precision=lax.Precision.HIGHEST)


def _full(a):
    return pl.BlockSpec(a.shape, lambda j: (0,) * a.ndim)


def _prologue(t, fw, emb128, ale_W, ale_b, h2, gid2, lmat, vpad1):
    def body(t_ref, fw_ref, emb_ref, aw_ref, ab_ref, h_ref, g_ref, lm_ref,
             vp_ref, nf_ref, lv_ref):
        xp = 2.0 * np.pi * _dot(t_ref[...], fw_ref[...])
        temb = jnp.concatenate([jnp.cos(xp), jnp.sin(xp)], axis=1)
        hh = h_ref[...]
        oh_h = (hh == lax.broadcasted_iota(i32, (NB, 128), 1)).astype(f32)
        nf_emb = _dot(oh_h, emb_ref[...])
        gg = g_ref[...]
        oh_g = (gg == lax.broadcasted_iota(i32, (NB, G), 1)).astype(f32)
        tpa = _dot(oh_g, temb)
        aw = aw_ref[...]
        nf = (_dot(nf_emb, aw[:128])
              + _dot(tpa, aw[128:])
              + ab_ref[...])
        nf_ref[...] = nf
        lv_ref[...] = _dot(oh_g, lm_ref[...]) + vp_ref[...]

    return pl.pallas_call(
        body,
        grid=(NP // NB,),
        in_specs=[
            _full(t), _full(fw), _full(emb128), _full(ale_W), _full(ale_b),
            pl.BlockSpec((NB, 1), lambda j: (j, 0)),
            pl.BlockSpec((NB, 1), lambda j: (j, 0)),
            _full(lmat),
            pl.BlockSpec((NB, 16), lambda j: (j, 0)),
        ],
        out_specs=[
            pl.BlockSpec((NB, D), lambda j: (j, 0)),
            pl.BlockSpec((NB, 16), lambda j: (j, 0)),
        ],
        out_shape=[
            jax.ShapeDtypeStruct((NP, D), f32),
            jax.ShapeDtypeStruct((NP, 16), f32),
        ],
    )(t, fw, emb128, ale_W, ale_b, h2, gid2, lmat, vpad1)


def _node_pre(nf, lv, lns, lnb, wh2, wnlv):
    def body(nf_ref, lv_ref, s_ref, b_ref, wh_ref, wl_ref, hf_ref, t2_ref):
        hf = _ln(nf_ref[...], s_ref[...], b_ref[...])
        hf_ref[...] = hf
        ab = (jnp.dot(hf, wh_ref[...], preferred_element_type=f32)
              + jnp.dot(lv_ref[...], wl_ref[...],
                        preferred_element_type=f32))
        t2_ref[0, :, :] = ab[:, :D]
        t2_ref[1, :, :] = ab[:, D:]

    return pl.pallas_call(
        body,
        grid=(NP // NB,),
        in_specs=[
            pl.BlockSpec((NB, D), lambda j: (j, 0)),
            pl.BlockSpec((NB, 16), lambda j: (j, 0)),
            _full(lns), _full(lnb), _full(wh2), _full(wnlv),
        ],
        out_specs=[
            pl.BlockSpec((NB, D), lambda j: (j, 0)),
            pl.BlockSpec((2, NB, D), lambda j: (0, j, 0)),
        ],
        out_shape=[
            jax.ShapeDtypeStruct((NP, D), f32),
            jax.ShapeDtypeStruct((2, NP, D), f32),
        ],
    )(nf, lv, lns, lnb, wh2, wnlv)


def _pd_prep(posg3, s16):
    def body(pg_ref, s_ref, out_ref):
        dvec = pg_ref[1, :, :] - pg_ref[0, :, :]
        e = jnp.dot(dvec, s_ref[...], preferred_element_type=f32)
        col = lax.broadcasted_iota(i32, (EB, 64), 1)
        out_ref[...] = (jnp.where(col < 30, jnp.sin(e), 0.0)
                        + jnp.where((col >= 32) & (col < 62), jnp.cos(e), 0.0))

    return pl.pallas_call(
        body,
        grid=(EP // EB,),
        in_specs=[
            pl.BlockSpec((2, EB, 16), lambda j: (0, j, 0)),
            _full(s16),
        ],
        out_specs=pl.BlockSpec((EB, 64), lambda j: (j, 0)),
        out_shape=jax.ShapeDtypeStruct((EP, 64), f32),
    )(posg3, s16)


def _edge(gat3, pdemb, wpd, w2, b2, off):
    ec = gat3.shape[1]
    def body(g_ref, pd_ref, wpd_ref, w2_ref, b2_ref, ef_ref):
        ga = g_ref[0, :, :]
        gb = g_ref[1, :, :]
        pre = (ga + gb
               + jnp.dot(pd_ref[...], wpd_ref[...],
                         preferred_element_type=f32))
        e1 = _silu(pre)
        z = jnp.dot(e1, w2_ref[...], preferred_element_type=f32) + b2_ref[...]
        ef_ref[...] = _silu(z)

    return pl.pallas_call(
        body,
        grid=(ec // EB,),
        in_specs=[
            pl.BlockSpec((2, EB, D), lambda j: (0, j, 0)),
            pl.BlockSpec((EB, 64), lambda j: (j + off, 0)),
            _full(wpd), _full(w2), _full(b2),
        ],
        out_specs=pl.BlockSpec((EB, D), lambda j: (j, 0)),
        out_shape=jax.ShapeDtypeStruct((ec, D), f32),
    )(gat3, pdemb, wpd, w2, b2)


def _node_post(aggp, aggq, cntp, hf, nf, w1h, w1a, b1, w2, b2):
    def body(ag_ref, aq_ref, c_ref, hf_ref, nf_ref, w1h_ref, w1a_ref, b1_ref,
             w2_ref, b2_ref, out_ref):
        c = c_ref[0, :, 0:1] + c_ref[1, :, 0:1]
        agg = ((ag_ref[0, :, :] + ag_ref[1, :, :])
               + (aq_ref[0, :, :] + aq_ref[1, :, :])) / jnp.maximum(c, 1.0)
        n1 = _silu(jnp.dot(hf_ref[...], w1h_ref[...],
                           preferred_element_type=f32)
                   + jnp.dot(agg, w1a_ref[...], preferred_element_type=f32)
                   + b1_ref[...])
        n2 = _silu(jnp.dot(n1, w2_ref[...], preferred_element_type=f32)
                   + b2_ref[...])
        out_ref[...] = nf_ref[...] + n2

    return pl.pallas_call(
        body,
        grid=(NP // NB,),
        in_specs=[
            pl.BlockSpec((2, NB, D), lambda j: (0, j, 0)),
            pl.BlockSpec((2, NB, D), lambda j: (0, j, 0)),
            pl.BlockSpec((2, NB, 16), lambda j: (0, j, 0)),
            pl.BlockSpec((NB, D), lambda j: (j, 0)),
            pl.BlockSpec((NB, D), lambda j: (j, 0)),
            _full(w1h), _full(w1a), _full(b1), _full(w2), _full(b2),
        ],
        out_specs=pl.BlockSpec((NB, D), lambda j: (j, 0)),
        out_shape=jax.ShapeDtypeStruct((NP, D), f32),
    )(aggp, aggq, cntp, hf, nf, w1h, w1a, b1, w2, b2)


def _node_mp(aggp, aggq, cntp, hf, nf, w1h, w1a, b1, w2, b2, lv, lns, lnb,
             wh2, wnlv):
    def body(ag_ref, aq_ref, c_ref, hf_ref, nf_ref, w1h_ref, w1a_ref, b1_ref,
             w2_ref, b2_ref, lv_ref, s_ref, b2s_ref, wh_ref, wl_ref,
             out_ref, hfn_ref, t2_ref):
        c = c_ref[0, :, 0:1] + c_ref[1, :, 0:1]
        agg = ((ag_ref[0, :, :] + ag_ref[1, :, :])
               + (aq_ref[0, :, :] + aq_ref[1, :, :])) / jnp.maximum(c, 1.0)
        n1 = _silu(jnp.dot(hf_ref[...], w1h_ref[...],
                           preferred_element_type=f32)
                   + jnp.dot(agg, w1a_ref[...], preferred_element_type=f32)
                   + b1_ref[...])
        n2 = _silu(jnp.dot(n1, w2_ref[...], preferred_element_type=f32)
                   + b2_ref[...])
        nfn = nf_ref[...] + n2
        out_ref[...] = nfn
        hfx = _ln(nfn, s_ref[...], b2s_ref[...])
        hfn_ref[...] = hfx
        ab = (jnp.dot(hfx, wh_ref[...], preferred_element_type=f32)
              + jnp.dot(lv_ref[...], wl_ref[...],
                        preferred_element_type=f32))

        def pack(x):
            xi = lax.bitcast_convert_type(x, i32)
            xr = (xi + 0x7fff + ((xi >> 16) & 1)) >> 16
            return (xr[:, :64] << 16) | (xr[:, 64:] & 0xffff)

        t2_ref[0, :, :] = pack(ab[:, :D])
        t2_ref[1, :, :] = pack(ab[:, D:])

    return pl.pallas_call(
        body,
        grid=(NP // NB,),
        in_specs=[
            pl.BlockSpec((2, NB, D), lambda j: (0, j, 0)),
            pl.BlockSpec((2, NB, D), lambda j: (0, j, 0)),
            pl.BlockSpec((2, NB, 16), lambda j: (0, j, 0)),
            pl.BlockSpec((NB, D), lambda j: (j, 0)),
            pl.BlockSpec((NB, D), lambda j: (j, 0)),
            _full(w1h), _full(w1a), _full(b1), _full(w2), _full(b2),
            pl.BlockSpec((NB, 16), lambda j: (j, 0)),
            _full(lns), _full(lnb), _full(wh2), _full(wnlv),
        ],
        out_specs=[
            pl.BlockSpec((NB, D), lambda j: (j, 0)),
            pl.BlockSpec((NB, D), lambda j: (j, 0)),
            pl.BlockSpec((2, NB, D // 2), lambda j: (0, j, 0)),
        ],
        out_shape=[
            jax.ShapeDtypeStruct((NP, D), f32),
            jax.ShapeDtypeStruct((NP, D), f32),
            jax.ShapeDtypeStruct((2, NP, D // 2), i32),
        ],
    )(aggp, aggq, cntp, hf, nf, w1h, w1a, b1, w2, b2, lv, lns, lnb, wh2,
      wnlv)


def _epi1(nf, gid2, fs, fb, ovw1, ovb1, ovw2p):
    def body(nf_ref, g_ref, fs_ref, fb_ref, w1_ref, b1_ref, w2_ref,
             ov_ref, sv_ref, gf_ref, cn_ref):
        nfn = _ln(nf_ref[...], fs_ref[...], fb_ref[...])
        ovh = _silu(_dot(nfn, w1_ref[...])
                    + b1_ref[...])
        ov = _dot(ovh, w2_ref[...])
        ov_ref[...] = ov
        oh = (g_ref[...] == lax.broadcasted_iota(i32, (NB, G), 1)).astype(f32)

        @pl.when(pl.program_id(0) == 0)
        def _():
            sv_ref[...] = jnp.zeros((G, D), f32)
            gf_ref[...] = jnp.zeros((G, D), f32)
            cn_ref[...] = jnp.zeros((G, D), f32)

        dn = (((0,), (0,)), ((), ()))
        sv_ref[...] += lax.dot_general(oh, ov, dn, preferred_element_type=f32, precision=lax.Precision.HIGHEST)
        gf_ref[...] += lax.dot_general(oh, nfn, dn, preferred_element_type=f32, precision=lax.Precision.HIGHEST)
        cn_ref[...] += lax.dot_general(oh, jnp.ones((NB, D), f32), dn,
                                       preferred_element_type=f32,
                                       precision=lax.Precision.HIGHEST)

    return pl.pallas_call(
        body,
        grid=(NP // NB,),
        in_specs=[
            pl.BlockSpec((NB, D), lambda j: (j, 0)),
            pl.BlockSpec((NB, 1), lambda j: (j, 0)),
            _full(fs), _full(fb), _full(ovw1), _full(ovb1), _full(ovw2p),
        ],
        out_specs=[
            pl.BlockSpec((NB, D), lambda j: (j, 0)),
            pl.BlockSpec((G, D), lambda j: (0, 0)),
            pl.BlockSpec((G, D), lambda j: (0, 0)),
            pl.BlockSpec((G, D), lambda j: (0, 0)),
        ],
        out_shape=[
            jax.ShapeDtypeStruct((NP, D), f32),
            jax.ShapeDtypeStruct((G, D), f32),
            jax.ShapeDtypeStruct((G, D), f32),
            jax.ShapeDtypeStruct((G, D), f32),
        ],
    )(nf, gid2, fs, fb, ovw1, ovb1, ovw2p)


def _epi2(sv, gf, cn, olwp):
    def body(sv_ref, gf_ref, cn_ref, ol_ref, mv_ref, gl_ref):
        c = jnp.maximum(cn_ref[...], 1.0)
        mv_ref[...] = sv_ref[...] / c
        gl_ref[...] = _dot(gf_ref[...] / c, ol_ref[...])

    return pl.pallas_call(
        body,
        grid=(1,),
        in_specs=[_full(sv), _full(gf), _full(cn), _full(olwp)],
        out_specs=[
            pl.BlockSpec((G, D), lambda j: (0, 0)),
            pl.BlockSpec((G, D), lambda j: (0, 0)),
        ],
        out_shape=[
            jax.ShapeDtypeStruct((G, D), f32),
            jax.ShapeDtypeStruct((G, D), f32),
        ],
    )(sv, gf, cn, olwp)


def _epi3(ov, gid2, mv):
    def body(ov_ref, g_ref, mv_ref, out_ref):
        oh = (g_ref[...] == lax.broadcasted_iota(i32, (NB, G), 1)).astype(f32)
        out_ref[...] = ov_ref[...] - _dot(oh, mv_ref[...])

    return pl.pallas_call(
        body,
        grid=(NP // NB,),
        in_specs=[
            pl.BlockSpec((NB, D), lambda j: (j, 0)),
            pl.BlockSpec((NB, 1), lambda j: (j, 0)),
            _full(mv),
        ],
        out_specs=pl.BlockSpec((NB, D), lambda j: (j, 0)),
        out_shape=jax.ShapeDtypeStruct((NP, D), f32),
    )(ov, gid2, mv)


def kernel(t, pos, v, l, emb_table, fourier_W, ale_W, ale_b, ln_s, ln_b, vW,
           vb, eW1, eb1, eW2, eb2, nW1, nb1, nW2, nb2, fln_s, fln_b, ovW1,
           ovb1, ovW2, olW, h, node_index, edge_node_index):
    src = edge_node_index[0].astype(i32)
    dst = edge_node_index[1].astype(i32)
    src_p = jnp.full((EP,), N, i32).at[:E].set(src)
    dst_p = jnp.zeros((EP,), i32).at[:E].set(dst)
    cat_idx = jnp.concatenate([src_p, dst_p + NP]).reshape(2 * EP // 128, 128)
    src_p2 = src_p.reshape(EP // 128, 128)

    EPH = EP // 2
    cat_idx_c = [
        jnp.concatenate([src_p[c * EPH:(c + 1) * EPH],
                         dst_p[c * EPH:(c + 1) * EPH] + NP])
        .reshape(2 * EPH // 128, 128)
        for c in range(2)
    ]
    src_p2_c = [src_p[c * EPH:(c + 1) * EPH].reshape(EPH // 128, 128)
                for c in range(2)]

    gid2 = jnp.full((NP, 1), G, i32).at[:N, 0].set(node_index.astype(i32))
    h2 = jnp.zeros((NP, 1), i32).at[:N, 0].set(h.astype(i32))

    emb128 = jnp.zeros((128, D), f32).at[:101].set(emb_table)
    vpad1 = (jnp.zeros((NP, 16), f32).at[:N, 8:11].set(v)
             .at[:, 15].set(1.0))
    lmat = jnp.zeros((G, 16), f32).at[:, 0:6].set(l)
    pos16 = jnp.zeros((NP, 16), f32).at[:N, 0:3].set(pos)
    pos2 = jnp.concatenate([pos16, pos16], axis=0)

    zeros_np = jnp.zeros((NP, D), f32)
    zeros16 = jnp.zeros((NP, 16), f32)
    ones16 = jnp.ones((128, 16), f32)

    freqs = 2.0 * np.pi * np.arange(NFREQ, dtype=np.float32)
    s_np = np.zeros((16, 64), np.float32)
    for c in range(3):
        s_np[c, c * NFREQ:(c + 1) * NFREQ] = freqs
        s_np[c, 32 + c * NFREQ:32 + (c + 1) * NFREQ] = freqs
    s16 = jnp.asarray(s_np)

    wh2_l, wnlv_l, wpd_l = [], [], []
    for i in range(4):
        w_hi = eW1[i, 0:128]
        w_hj = eW1[i, 128:256]
        w_l = eW1[i, 256:262]
        w_v = eW1[i, 262:322]
        w_pd = eW1[i, 322:382]
        mv = vW[i] @ w_v
        bias_a = eb1[i] + vb[i] @ w_v
        wh2_l.append(jnp.concatenate([w_hi, w_hj], axis=1))
        wnlv = jnp.zeros((16, 2 * D), f32)
        wnlv = wnlv.at[0:6, :D].set(w_l)
        wnlv = wnlv.at[8:11, :D].set(-mv).at[8:11, D:].set(mv)
        wnlv = wnlv.at[15, :D].set(bias_a)
        wnlv_l.append(wnlv)
        wpd = jnp.zeros((64, D), f32)
        wpd = wpd.at[0:30].set(w_pd[0:30]).at[32:62].set(w_pd[30:60])
        wpd_l.append(wpd)

    ovw2p = jnp.zeros((D, D), f32).at[:, 0:3].set(ovW2)
    olwp = jnp.zeros((D, D), f32).at[:, 0:6].set(olW)

    nf, lv = _prologue(t, fourier_W, emb128, ale_W, ale_b, h2, gid2, lmat,
                       vpad1)
    posg = _sc_gather(pos2, cat_idx, 16).reshape(2, EP, 16)
    pdemb = _pd_prep(posg, s16)
    cntp = _sc_counts(src_p2, ones16, zeros16)

    for i in range(4):
        hf, t2 = _node_pre(nf, lv, ln_s[i].reshape(1, D),
                           ln_b[i].reshape(1, D), wh2_l[i], wnlv_l[i])
        t2r = t2.reshape(2 * NP, D)
        gat0 = _sc_gather(t2r, cat_idx_c[0], D).reshape(2, EPH, D)
        ef0 = _edge(gat0, pdemb, wpd_l[i], eW2[i], eb2[i].reshape(1, D), 0)
        gat1 = _sc_gather(t2r, cat_idx_c[1], D).reshape(2, EPH, D)
        agg0 = _sc_scatter_add(ef0, src_p2_c[0], zeros_np)
        ef1 = _edge(gat1, pdemb, wpd_l[i], eW2[i], eb2[i].reshape(1, D),
                    EPH // EB)
        agg1 = _sc_scatter_add(ef1, src_p2_c[1], zeros_np)
        nf = _node_post(agg0, agg1, cntp, hf, nf, nW1[i, :D], nW1[i, D:],
                        nb1[i].reshape(1, D), nW2[i], nb2[i].reshape(1, D))

    ov, sv, gf, cn = _epi1(nf, gid2, fln_s.reshape(1, D),
                           fln_b.reshape(1, D), ovW1, ovb1.reshape(1, D),
                           ovw2p)
    mv_g, gl = _epi2(sv, gf, cn, olwp)
    ovc = _epi3(ov, gid2, mv_g)
    return ovc[:N, 0:3], gl[:, 0:6]
